# Initial kernel scaffold; baseline (speedup 1.0000x reference)
import jax
import jax.numpy as jnp
from jax.experimental import pallas as pl


def kernel(species, edge_index, edge_attr, edge_vec, t, ea_W, ea_b, eb_W1, eb_b1, eb_W2, eb_b2, ps_W1, ps_b1, ps_W2, ps_b2, ph_W1, ph_b1, ph_W2, ph_b2, pv_W1, pv_b1, pv_W2, pv_b2, rff_W, tm_W1, tm_b1, tm_W2, tm_b2):
    raise NotImplementedError("write your pallas kernel here")


# speedup vs baseline: 5.6422x; 5.6422x over previous
import functools

import jax
import jax.numpy as jnp
from jax import lax
from jax.experimental import pallas as pl
from jax.experimental.pallas import tpu as pltpu

N = 10000
E = 160000
D = 128

BN = 2000
BE = 2000


def _silu(x):
    return x * jax.nn.sigmoid(x)


def _nodes_pre_body(species, t, ea_W, ea_b, rff_W, tm_W1, tm_b1, tm_W2, tm_b2,
                    f_out, te_out):
    f_out[...] = (jnp.dot(species[...], ea_W[...],
                          preferred_element_type=jnp.float32) + ea_b[...])
    proj = (2.0 * jnp.pi) * (t[...] * rff_W[...])
    feats = jnp.concatenate([jnp.cos(proj), jnp.sin(proj)], axis=-1)
    u = _silu(jnp.dot(feats, tm_W1[...], preferred_element_type=jnp.float32)
              + tm_b1[...])
    te_out[...] = (jnp.dot(u, tm_W2[...], preferred_element_type=jnp.float32)
                   + tm_b2[...])


def _nodes_pre(species, t, ea_W, ea_b, rff_W, tm_W1, tm_b1, tm_W2, tm_b2):
    nb = N // BN
    full = lambda shape: pl.BlockSpec(shape, lambda n: (0,) * len(shape))
    return pl.pallas_call(
        _nodes_pre_body,
        grid=(nb,),
        in_specs=[
            pl.BlockSpec((BN, 100), lambda n: (n, 0)),
            pl.BlockSpec((BN, 1), lambda n: (n, 0)),
            full((100, D)), full((1, D)), full((1, D // 2)),
            full((D, D)), full((1, D)), full((D, D)), full((1, D)),
        ],
        out_specs=[
            pl.BlockSpec((BN, D), lambda n: (n, 0)),
            pl.BlockSpec((BN, D), lambda n: (n, 0)),
        ],
        out_shape=[
            jax.ShapeDtypeStruct((N, D), jnp.float32),
            jax.ShapeDtypeStruct((N, D), jnp.float32),
        ],
    )(species, t, ea_W, ea_b, rff_W, tm_W1, tm_b1, tm_W2, tm_b2)


def _ea_body(edge_attr, W1, b1, W2, b2, ea_out):
    u = _silu(jnp.dot(edge_attr[...], W1[...],
                      preferred_element_type=jnp.float32) + b1[...])
    ea_out[...] = (jnp.dot(u, W2[...], preferred_element_type=jnp.float32)
                   + b2[...])


def _ea_mlp(edge_attr, W1, b1, W2, b2):
    nb = E // BE
    full = lambda shape: pl.BlockSpec(shape, lambda n: (0,) * len(shape))
    return pl.pallas_call(
        _ea_body,
        grid=(nb,),
        in_specs=[
            pl.BlockSpec((BE, 120), lambda n: (n, 0)),
            full((120, D)), full((1, D)), full((D, D)), full((1, D)),
        ],
        out_specs=pl.BlockSpec((BE, D), lambda n: (n, 0)),
        out_shape=jax.ShapeDtypeStruct((E, D), jnp.float32),
    )(edge_attr, W1, b1, W2, b2)


def _edges_body(fi, fj, ea, ev, ps_W1, ps_b1, ps_W2, ps_b2,
                pv_W1, pv_b1, pv_W2, pv_b2, out):
    fi_v = fi[...]
    fj_v = fj[...]
    ea_v = ea[...]

    def mlp3(W1, b1, W2, b2):
        u = (jnp.dot(fi_v, W1[0:D], preferred_element_type=jnp.float32)
             + jnp.dot(fj_v, W1[D:2 * D], preferred_element_type=jnp.float32)
             + jnp.dot(ea_v, W1[2 * D:3 * D], preferred_element_type=jnp.float32)
             + b1[...])
        return jnp.dot(_silu(u), W2[...],
                       preferred_element_type=jnp.float32) + b2[...]

    msg = mlp3(ps_W1, ps_b1, ps_W2, ps_b2) * fi_v
    vw = mlp3(pv_W1, pv_b1, pv_W2, pv_b2)
    ev_v = ev[...]
    out[0] = msg
    out[1] = vw * ev_v[:, 0:1]
    out[2] = vw * ev_v[:, 1:2]
    out[3] = vw * ev_v[:, 2:3]


def _edges_mlp(fi, fj, ea, edge_vec, ps_W1, ps_b1, ps_W2, ps_b2,
               pv_W1, pv_b1, pv_W2, pv_b2):
    nb = E // BE
    full = lambda shape: pl.BlockSpec(shape, lambda n: (0,) * len(shape))
    return pl.pallas_call(
        _edges_body,
        grid=(nb,),
        in_specs=[
            pl.BlockSpec((BE, D), lambda n: (n, 0)),
            pl.BlockSpec((BE, D), lambda n: (n, 0)),
            pl.BlockSpec((BE, D), lambda n: (n, 0)),
            pl.BlockSpec((BE, 3), lambda n: (n, 0)),
            full((3 * D, D)), full((1, D)), full((D, D)), full((1, D)),
            full((3 * D, D)), full((1, D)), full((D, D)), full((1, D)),
        ],
        out_specs=pl.BlockSpec((4, BE, D), lambda n: (0, n, 0)),
        out_shape=jax.ShapeDtypeStruct((4, E, D), jnp.float32),
    )(fi, fj, ea, edge_vec, ps_W1, ps_b1, ps_W2, ps_b2,
      pv_W1, pv_b1, pv_W2, pv_b2)


def _h0_body(f, agg, te, W1, b1, W2, b2, h0_out):
    u = (jnp.dot(f[...], W1[0:D], preferred_element_type=jnp.float32)
         + jnp.dot(agg[...], W1[D:2 * D], preferred_element_type=jnp.float32)
         + b1[...])
    h0_out[...] = (jnp.dot(_silu(u), W2[...],
                           preferred_element_type=jnp.float32)
                   + b2[...] + te[...])


def _h0_mlp(f, agg, te, W1, b1, W2, b2):
    nb = N // BN
    full = lambda shape: pl.BlockSpec(shape, lambda n: (0,) * len(shape))
    return pl.pallas_call(
        _h0_body,
        grid=(nb,),
        in_specs=[
            pl.BlockSpec((BN, D), lambda n: (n, 0)),
            pl.BlockSpec((BN, D), lambda n: (n, 0)),
            pl.BlockSpec((BN, D), lambda n: (n, 0)),
            full((2 * D, D)), full((1, D)), full((D, D)), full((1, D)),
        ],
        out_specs=pl.BlockSpec((BN, D), lambda n: (n, 0)),
        out_shape=jax.ShapeDtypeStruct((N, D), jnp.float32),
    )(f, agg, te, W1, b1, W2, b2)


def kernel(species, edge_index, edge_attr, edge_vec, t,
           ea_W, ea_b, eb_W1, eb_b1, eb_W2, eb_b2,
           ps_W1, ps_b1, ps_W2, ps_b2,
           ph_W1, ph_b1, ph_W2, ph_b2,
           pv_W1, pv_b1, pv_W2, pv_b2,
           rff_W, tm_W1, tm_b1, tm_W2, tm_b2):
    r = lambda b: b.reshape(1, -1)
    i = edge_index[0]
    j = edge_index[1]

    f, te = _nodes_pre(species, t, ea_W, r(ea_b), rff_W,
                       tm_W1, r(tm_b1), tm_W2, r(tm_b2))
    ea = _ea_mlp(edge_attr, eb_W1, r(eb_b1), eb_W2, r(eb_b2))

    fi = jnp.take(f, i, axis=0)
    fj = jnp.take(f, j, axis=0)

    stacked = _edges_mlp(fi, fj, ea, edge_vec,
                         ps_W1, r(ps_b1), ps_W2, r(ps_b2),
                         pv_W1, r(pv_b1), pv_W2, r(pv_b2))

    segs = jax.ops.segment_sum(
        stacked.reshape(4 * E, D),
        jnp.tile(j, 4) + jnp.repeat(jnp.arange(4, dtype=jnp.int32) * N, E),
        num_segments=4 * N).reshape(4, N, D)
    agg = segs[0]
    v0 = jnp.transpose(segs[1:4], (1, 2, 0))

    h0 = _h0_mlp(f, agg, te, ph_W1, r(ph_b1), ph_W2, r(ph_b2))
    return (h0, v0, ea)


# speedup vs baseline: 7.0414x; 1.2480x over previous
import functools

import jax
import jax.numpy as jnp
from jax import lax
from jax.experimental import pallas as pl
from jax.experimental.pallas import tpu as pltpu
from jax.experimental.pallas import tpu_sc as plsc

N = 10000
E = 160000
D = 128

BN = 2000
BE = 2000

G = 128
NW = 32
NGROUPS = E // G
GPAD = ((NGROUPS + NW - 1) // NW) * NW
EPAD = GPAD * G
GPW = GPAD // NW


def _silu(x):
    return x * jax.nn.sigmoid(x)


def _nodes_pre_body(species, t, ea_W, ea_b, rff_W, tm_W1, tm_b1, tm_W2, tm_b2,
                    f_out, te_out):
    f_out[...] = (jnp.dot(species[...], ea_W[...],
                          preferred_element_type=jnp.float32) + ea_b[...])
    proj = (2.0 * jnp.pi) * (t[...] * rff_W[...])
    feats = jnp.concatenate([jnp.cos(proj), jnp.sin(proj)], axis=-1)
    u = _silu(jnp.dot(feats, tm_W1[...], preferred_element_type=jnp.float32)
              + tm_b1[...])
    te_out[...] = (jnp.dot(u, tm_W2[...], preferred_element_type=jnp.float32)
                   + tm_b2[...])


def _nodes_pre(species, t, ea_W, ea_b, rff_W, tm_W1, tm_b1, tm_W2, tm_b2):
    nb = N // BN
    full = lambda shape: pl.BlockSpec(shape, lambda n: (0,) * len(shape))
    return pl.pallas_call(
        _nodes_pre_body,
        grid=(nb,),
        in_specs=[
            pl.BlockSpec((BN, 100), lambda n: (n, 0)),
            pl.BlockSpec((BN, 1), lambda n: (n, 0)),
            full((100, D)), full((1, D)), full((1, D // 2)),
            full((D, D)), full((1, D)), full((D, D)), full((1, D)),
        ],
        out_specs=[
            pl.BlockSpec((BN, D), lambda n: (n, 0)),
            pl.BlockSpec((BN, D), lambda n: (n, 0)),
        ],
        out_shape=[
            jax.ShapeDtypeStruct((N, D), jnp.float32),
            jax.ShapeDtypeStruct((N, D), jnp.float32),
        ],
    )(species, t, ea_W, ea_b, rff_W, tm_W1, tm_b1, tm_W2, tm_b2)


def _ea_body(edge_attr, W1, b1, W2, b2, ea_out):
    u = _silu(jnp.dot(edge_attr[...], W1[...],
                      preferred_element_type=jnp.float32) + b1[...])
    ea_out[...] = (jnp.dot(u, W2[...], preferred_element_type=jnp.float32)
                   + b2[...])


def _ea_mlp(edge_attr, W1, b1, W2, b2):
    nb = E // BE
    full = lambda shape: pl.BlockSpec(shape, lambda n: (0,) * len(shape))
    return pl.pallas_call(
        _ea_body,
        grid=(nb,),
        in_specs=[
            pl.BlockSpec((BE, 120), lambda n: (n, 0)),
            full((120, D)), full((1, D)), full((D, D)), full((1, D)),
        ],
        out_specs=pl.BlockSpec((BE, D), lambda n: (n, 0)),
        out_shape=jax.ShapeDtypeStruct((E, D), jnp.float32),
    )(edge_attr, W1, b1, W2, b2)


def _edges_body(fi, fj, ea, ev, ps_W1, ps_b1, ps_W2, ps_b2,
                pv_W1, pv_b1, pv_W2, pv_b2, out):
    fi_v = fi[...]
    fj_v = fj[...]
    ea_v = ea[...]

    def mlp3(W1, b1, W2, b2):
        u = (jnp.dot(fi_v, W1[0:D], preferred_element_type=jnp.float32)
             + jnp.dot(fj_v, W1[D:2 * D], preferred_element_type=jnp.float32)
             + jnp.dot(ea_v, W1[2 * D:3 * D], preferred_element_type=jnp.float32)
             + b1[...])
        return jnp.dot(_silu(u), W2[...],
                       preferred_element_type=jnp.float32) + b2[...]

    msg = mlp3(ps_W1, ps_b1, ps_W2, ps_b2) * fi_v
    vw = mlp3(pv_W1, pv_b1, pv_W2, pv_b2)
    ev_v = ev[...]
    out[0] = msg
    out[1] = vw * ev_v[:, 0:1]
    out[2] = vw * ev_v[:, 1:2]
    out[3] = vw * ev_v[:, 2:3]


def _edges_mlp(fi, fj, ea, edge_vec, ps_W1, ps_b1, ps_W2, ps_b2,
               pv_W1, pv_b1, pv_W2, pv_b2):
    nb = E // BE
    full = lambda shape: pl.BlockSpec(shape, lambda n: (0,) * len(shape))
    return pl.pallas_call(
        _edges_body,
        grid=(nb,),
        in_specs=[
            pl.BlockSpec((BE, D), lambda n: (n, 0)),
            pl.BlockSpec((BE, D), lambda n: (n, 0)),
            pl.BlockSpec((BE, D), lambda n: (n, 0)),
            pl.BlockSpec((BE, 3), lambda n: (n, 0)),
            full((3 * D, D)), full((1, D)), full((D, D)), full((1, D)),
            full((3 * D, D)), full((1, D)), full((D, D)), full((1, D)),
        ],
        out_specs=pl.BlockSpec((4, BE, D), lambda n: (0, n, 0)),
        out_shape=jax.ShapeDtypeStruct((4, E, D), jnp.float32),
    )(fi, fj, ea, edge_vec, ps_W1, ps_b1, ps_W2, ps_b2,
      pv_W1, pv_b1, pv_W2, pv_b2)


def _h0_body(f, agg, te, W1, b1, W2, b2, h0_out):
    u = (jnp.dot(f[...], W1[0:D], preferred_element_type=jnp.float32)
         + jnp.dot(agg[...], W1[D:2 * D], preferred_element_type=jnp.float32)
         + b1[...])
    h0_out[...] = (jnp.dot(_silu(u), W2[...],
                           preferred_element_type=jnp.float32)
                   + b2[...] + te[...])


def _h0_mlp(f, agg, te, W1, b1, W2, b2):
    nb = N // BN
    full = lambda shape: pl.BlockSpec(shape, lambda n: (0,) * len(shape))
    return pl.pallas_call(
        _h0_body,
        grid=(nb,),
        in_specs=[
            pl.BlockSpec((BN, D), lambda n: (n, 0)),
            pl.BlockSpec((BN, D), lambda n: (n, 0)),
            pl.BlockSpec((BN, D), lambda n: (n, 0)),
            full((2 * D, D)), full((1, D)), full((D, D)), full((1, D)),
        ],
        out_specs=pl.BlockSpec((BN, D), lambda n: (n, 0)),
        out_shape=jax.ShapeDtypeStruct((N, D), jnp.float32),
    )(f, agg, te, W1, b1, W2, b2)


def _sc_gather(f, ig, jg):
    mesh = plsc.VectorSubcoreMesh(core_axis_name="c", subcore_axis_name="s")

    @functools.partial(
        pl.kernel,
        mesh=mesh,
        out_type=[jax.ShapeDtypeStruct((EPAD, D), jnp.float32),
                  jax.ShapeDtypeStruct((EPAD, D), jnp.float32)],
        scratch_types=[
            pltpu.VMEM((GPW, G), jnp.int32),
            pltpu.VMEM((GPW, G), jnp.int32),
            pltpu.VMEM((G, D), jnp.float32),
            pltpu.VMEM((G, D), jnp.float32),
            pltpu.SemaphoreType.DMA,
            pltpu.SemaphoreType.DMA,
        ],
    )
    def k(f_hbm, ig_hbm, jg_hbm, fi_hbm, fj_hbm, iv, jv, ri, rj, si, sj):
        c = lax.axis_index("c")
        s = lax.axis_index("s")
        wid = s * 2 + c
        start = wid * GPW
        pltpu.sync_copy(ig_hbm.at[pl.ds(start, GPW)], iv)
        pltpu.sync_copy(jg_hbm.at[pl.ds(start, GPW)], jv)

        def body(kk, carry):
            g = start + kk
            cp1 = pltpu.async_copy(f_hbm.at[iv.at[kk]], ri, si)
            cp2 = pltpu.async_copy(f_hbm.at[jv.at[kk]], rj, sj)
            cp1.wait()
            cp2.wait()
            pltpu.sync_copy(ri, fi_hbm.at[pl.ds(g * G, G)])
            pltpu.sync_copy(rj, fj_hbm.at[pl.ds(g * G, G)])
            return carry

        lax.fori_loop(0, GPW, body, 0)

    return k(f, ig, jg)


def kernel(species, edge_index, edge_attr, edge_vec, t,
           ea_W, ea_b, eb_W1, eb_b1, eb_W2, eb_b2,
           ps_W1, ps_b1, ps_W2, ps_b2,
           ph_W1, ph_b1, ph_W2, ph_b2,
           pv_W1, pv_b1, pv_W2, pv_b2,
           rff_W, tm_W1, tm_b1, tm_W2, tm_b2):
    r = lambda b: b.reshape(1, -1)
    i = edge_index[0]
    j = edge_index[1]

    f, te = _nodes_pre(species, t, ea_W, r(ea_b), rff_W,
                       tm_W1, r(tm_b1), tm_W2, r(tm_b2))
    ea = _ea_mlp(edge_attr, eb_W1, r(eb_b1), eb_W2, r(eb_b2))

    pad = EPAD - E
    ig = jnp.pad(i, (0, pad)).reshape(GPAD, G)
    jg = jnp.pad(j, (0, pad)).reshape(GPAD, G)
    fi, fj = _sc_gather(f, ig, jg)

    stacked = _edges_mlp(fi, fj, ea, edge_vec,
                         ps_W1, r(ps_b1), ps_W2, r(ps_b2),
                         pv_W1, r(pv_b1), pv_W2, r(pv_b2))

    segs = jax.ops.segment_sum(
        stacked.reshape(4 * E, D),
        jnp.tile(j, 4) + jnp.repeat(jnp.arange(4, dtype=jnp.int32) * N, E),
        num_segments=4 * N).reshape(4, N, D)
    agg = segs[0]
    v0 = jnp.transpose(segs[1:4], (1, 2, 0))

    h0 = _h0_mlp(f, agg, te, ph_W1, r(ph_b1), ph_W2, r(ph_b2))
    return (h0, v0, ea)


# speedup vs baseline: 14.9272x; 2.1199x over previous
import functools

import jax
import jax.numpy as jnp
from jax import lax
from jax.experimental import pallas as pl
from jax.experimental.pallas import tpu as pltpu
from jax.experimental.pallas import tpu_sc as plsc

N = 10000
E = 160000
D = 128

BN = 2000
BE = 2000

G = 128
NW = 32
NGROUPS = E // G
GPAD = ((NGROUPS + NW - 1) // NW) * NW
EPAD = GPAD * G
GPW = GPAD // NW


def _silu(x):
    return x * jax.nn.sigmoid(x)


def _nodes_pre_body(species, t, ea_W, ea_b, rff_W, tm_W1, tm_b1, tm_W2, tm_b2,
                    f_out, te_out):
    f_out[...] = (jnp.dot(species[...], ea_W[...],
                          preferred_element_type=jnp.float32) + ea_b[...])
    proj = (2.0 * jnp.pi) * (t[...] * rff_W[...])
    feats = jnp.concatenate([jnp.cos(proj), jnp.sin(proj)], axis=-1)
    u = _silu(jnp.dot(feats, tm_W1[...], preferred_element_type=jnp.float32)
              + tm_b1[...])
    te_out[...] = (jnp.dot(u, tm_W2[...], preferred_element_type=jnp.float32)
                   + tm_b2[...])


def _nodes_pre(species, t, ea_W, ea_b, rff_W, tm_W1, tm_b1, tm_W2, tm_b2):
    nb = N // BN
    full = lambda shape: pl.BlockSpec(shape, lambda n: (0,) * len(shape))
    return pl.pallas_call(
        _nodes_pre_body,
        grid=(nb,),
        in_specs=[
            pl.BlockSpec((BN, 100), lambda n: (n, 0)),
            pl.BlockSpec((BN, 1), lambda n: (n, 0)),
            full((100, D)), full((1, D)), full((1, D // 2)),
            full((D, D)), full((1, D)), full((D, D)), full((1, D)),
        ],
        out_specs=[
            pl.BlockSpec((BN, D), lambda n: (n, 0)),
            pl.BlockSpec((BN, D), lambda n: (n, 0)),
        ],
        out_shape=[
            jax.ShapeDtypeStruct((N, D), jnp.float32),
            jax.ShapeDtypeStruct((N, D), jnp.float32),
        ],
    )(species, t, ea_W, ea_b, rff_W, tm_W1, tm_b1, tm_W2, tm_b2)


def _ea_body(edge_attr, W1, b1, W2, b2, ea_out):
    u = _silu(jnp.dot(edge_attr[...], W1[...],
                      preferred_element_type=jnp.float32) + b1[...])
    ea_out[...] = (jnp.dot(u, W2[...], preferred_element_type=jnp.float32)
                   + b2[...])


def _ea_mlp(edge_attr, W1, b1, W2, b2):
    nb = E // BE
    full = lambda shape: pl.BlockSpec(shape, lambda n: (0,) * len(shape))
    return pl.pallas_call(
        _ea_body,
        grid=(nb,),
        in_specs=[
            pl.BlockSpec((BE, 120), lambda n: (n, 0)),
            full((120, D)), full((1, D)), full((D, D)), full((1, D)),
        ],
        out_specs=pl.BlockSpec((BE, D), lambda n: (n, 0)),
        out_shape=jax.ShapeDtypeStruct((E, D), jnp.float32),
    )(edge_attr, W1, b1, W2, b2)


def _edges_body(fi, fj, ea, ev, ps_W1, ps_b1, ps_W2, ps_b2,
                pv_W1, pv_b1, pv_W2, pv_b2, out):
    fi_v = fi[...]
    fj_v = fj[...]
    ea_v = ea[...]

    def mlp3(W1, b1, W2, b2):
        u = (jnp.dot(fi_v, W1[0:D], preferred_element_type=jnp.float32)
             + jnp.dot(fj_v, W1[D:2 * D], preferred_element_type=jnp.float32)
             + jnp.dot(ea_v, W1[2 * D:3 * D], preferred_element_type=jnp.float32)
             + b1[...])
        return jnp.dot(_silu(u), W2[...],
                       preferred_element_type=jnp.float32) + b2[...]

    msg = mlp3(ps_W1, ps_b1, ps_W2, ps_b2) * fi_v
    vw = mlp3(pv_W1, pv_b1, pv_W2, pv_b2)
    ev_v = ev[...]
    out[0] = msg
    out[1] = vw * ev_v[:, 0:1]
    out[2] = vw * ev_v[:, 1:2]
    out[3] = vw * ev_v[:, 2:3]


def _edges_mlp(fi, fj, ea, edge_vec, ps_W1, ps_b1, ps_W2, ps_b2,
               pv_W1, pv_b1, pv_W2, pv_b2):
    nb = E // BE
    full = lambda shape: pl.BlockSpec(shape, lambda n: (0,) * len(shape))
    return pl.pallas_call(
        _edges_body,
        grid=(nb,),
        in_specs=[
            pl.BlockSpec((BE, D), lambda n: (n, 0)),
            pl.BlockSpec((BE, D), lambda n: (n, 0)),
            pl.BlockSpec((BE, D), lambda n: (n, 0)),
            pl.BlockSpec((BE, 3), lambda n: (n, 0)),
            full((3 * D, D)), full((1, D)), full((D, D)), full((1, D)),
            full((3 * D, D)), full((1, D)), full((D, D)), full((1, D)),
        ],
        out_specs=pl.BlockSpec((4, BE, D), lambda n: (0, n, 0)),
        out_shape=jax.ShapeDtypeStruct((4, E, D), jnp.float32),
    )(fi, fj, ea, edge_vec, ps_W1, ps_b1, ps_W2, ps_b2,
      pv_W1, pv_b1, pv_W2, pv_b2)


def _h0_body(f, agg, te, W1, b1, W2, b2, h0_out):
    u = (jnp.dot(f[...], W1[0:D], preferred_element_type=jnp.float32)
         + jnp.dot(agg[...], W1[D:2 * D], preferred_element_type=jnp.float32)
         + b1[...])
    h0_out[...] = (jnp.dot(_silu(u), W2[...],
                           preferred_element_type=jnp.float32)
                   + b2[...] + te[...])


def _h0_mlp(f, agg, te, W1, b1, W2, b2):
    nb = N // BN
    full = lambda shape: pl.BlockSpec(shape, lambda n: (0,) * len(shape))
    return pl.pallas_call(
        _h0_body,
        grid=(nb,),
        in_specs=[
            pl.BlockSpec((BN, D), lambda n: (n, 0)),
            pl.BlockSpec((BN, D), lambda n: (n, 0)),
            pl.BlockSpec((BN, D), lambda n: (n, 0)),
            full((2 * D, D)), full((1, D)), full((D, D)), full((1, D)),
        ],
        out_specs=pl.BlockSpec((BN, D), lambda n: (n, 0)),
        out_shape=jax.ShapeDtypeStruct((N, D), jnp.float32),
    )(f, agg, te, W1, b1, W2, b2)


def _sc_gather(f, ig, jg):
    mesh = plsc.VectorSubcoreMesh(core_axis_name="c", subcore_axis_name="s")

    @functools.partial(
        pl.kernel,
        mesh=mesh,
        out_type=[jax.ShapeDtypeStruct((EPAD, D), jnp.float32),
                  jax.ShapeDtypeStruct((EPAD, D), jnp.float32)],
        scratch_types=[
            pltpu.VMEM((GPW, G), jnp.int32),
            pltpu.VMEM((GPW, G), jnp.int32),
            pltpu.VMEM((G, D), jnp.float32),
            pltpu.VMEM((G, D), jnp.float32),
            pltpu.SemaphoreType.DMA,
            pltpu.SemaphoreType.DMA,
        ],
    )
    def k(f_hbm, ig_hbm, jg_hbm, fi_hbm, fj_hbm, iv, jv, ri, rj, si, sj):
        c = lax.axis_index("c")
        s = lax.axis_index("s")
        wid = s * 2 + c
        start = wid * GPW
        pltpu.sync_copy(ig_hbm.at[pl.ds(start, GPW)], iv)
        pltpu.sync_copy(jg_hbm.at[pl.ds(start, GPW)], jv)

        def body(kk, carry):
            g = start + kk
            cp1 = pltpu.async_copy(f_hbm.at[iv.at[kk]], ri, si)
            cp2 = pltpu.async_copy(f_hbm.at[jv.at[kk]], rj, sj)
            cp1.wait()
            cp2.wait()
            pltpu.sync_copy(ri, fi_hbm.at[pl.ds(g * G, G)])
            pltpu.sync_copy(rj, fj_hbm.at[pl.ds(g * G, G)])
            return carry

        lax.fori_loop(0, GPW, body, 0)

    return k(f, ig, jg)


NSUB = 16
GPS = NGROUPS // NSUB + 1
GWIN = 88
NPAD = 10240
NROWS = NPAD // NSUB
ZR = 128


def _sc_scatter(stacked, jg):
    mesh = plsc.VectorSubcoreMesh(core_axis_name="c", subcore_axis_name="s")

    @functools.partial(
        pl.kernel,
        mesh=mesh,
        out_type=jax.ShapeDtypeStruct((4, NPAD, D), jnp.float32),
        scratch_types=[
            pltpu.VMEM((GWIN, G), jnp.int32),
            pltpu.VMEM((G, D), jnp.float32),
            pltpu.VMEM((ZR, D), jnp.float32),
            pltpu.VMEM_SHARED((NPAD, D), jnp.float32),
        ],
    )
    def k(st_hbm, jg_hbm, out_hbm, idxv, dbuf, zbuf, acc):
        c = lax.axis_index("c")
        s = lax.axis_index("s")

        def zb(tt, carry):
            zbuf[tt // 8, pl.ds((tt % 8) * 16, 16)] = jnp.zeros((16,),
                                                                jnp.float32)
            return carry

        lax.fori_loop(0, ZR * 8, zb, 0)

        rem = NGROUPS - NSUB * (GPS - 1)
        start = s * (GPS - 1) + jnp.minimum(s, rem)
        count = (GPS - 1) + (s < rem).astype(jnp.int32)
        astart = pl.multiple_of((start // 8) * 8, 8)
        off = start - astart
        pltpu.sync_copy(jg_hbm.at[pl.ds(astart, GWIN)], idxv)

        for p in range(2):
            pass_idx = c * 2 + p
            for q in range(NROWS // ZR):
                pltpu.sync_copy(zbuf,
                                acc.at[pl.ds(s * NROWS + q * ZR, ZR)])
            plsc.subcore_barrier()

            def body(kk, carry):
                @pl.when(kk < count)
                def _():
                    g = start + kk
                    pltpu.sync_copy(st_hbm.at[pass_idx, pl.ds(g * G, G)],
                                    dbuf)
                    pltpu.sync_copy(dbuf, acc.at[idxv.at[off + kk]],
                                    add=True)
                return carry

            lax.fori_loop(0, GPS, body, 0)
            plsc.subcore_barrier()
            pltpu.sync_copy(acc.at[pl.ds(s * NROWS, NROWS)],
                            out_hbm.at[pass_idx, pl.ds(s * NROWS, NROWS)])
            plsc.subcore_barrier()

    return k(stacked, jg)


def kernel(species, edge_index, edge_attr, edge_vec, t,
           ea_W, ea_b, eb_W1, eb_b1, eb_W2, eb_b2,
           ps_W1, ps_b1, ps_W2, ps_b2,
           ph_W1, ph_b1, ph_W2, ph_b2,
           pv_W1, pv_b1, pv_W2, pv_b2,
           rff_W, tm_W1, tm_b1, tm_W2, tm_b2):
    r = lambda b: b.reshape(1, -1)
    i = edge_index[0]
    j = edge_index[1]

    f, te = _nodes_pre(species, t, ea_W, r(ea_b), rff_W,
                       tm_W1, r(tm_b1), tm_W2, r(tm_b2))
    ea = _ea_mlp(edge_attr, eb_W1, r(eb_b1), eb_W2, r(eb_b2))

    pad = EPAD - E
    ig = jnp.pad(i, (0, pad)).reshape(GPAD, G)
    jg = jnp.pad(j, (0, pad)).reshape(GPAD, G)
    fi, fj = _sc_gather(f, ig, jg)

    stacked = _edges_mlp(fi, fj, ea, edge_vec,
                         ps_W1, r(ps_b1), ps_W2, r(ps_b2),
                         pv_W1, r(pv_b1), pv_W2, r(pv_b2))

    segs = _sc_scatter(stacked, jg)
    agg = segs[0]
    v0 = jnp.transpose(segs[1:4, :N], (1, 2, 0))

    h0 = _h0_mlp(f, agg, te, ph_W1, r(ph_b1), ph_W2, r(ph_b2))
    return (h0, v0, ea)


# speedup vs baseline: 15.4670x; 1.0362x over previous
import functools

import jax
import jax.numpy as jnp
from jax import lax
from jax.experimental import pallas as pl
from jax.experimental.pallas import tpu as pltpu
from jax.experimental.pallas import tpu_sc as plsc

N = 10000
E = 160000
D = 128

BN = 2000
BE = 2000

G = 128
NW = 32
NGROUPS = E // G
GPAD = ((NGROUPS + NW - 1) // NW) * NW
EPAD = GPAD * G
GPW = GPAD // NW


def _silu(x):
    return x * jax.nn.sigmoid(x)


def _nodes_pre_body(species, t, ea_W, ea_b, rff_W, tm_W1, tm_b1, tm_W2, tm_b2,
                    f_out, te_out):
    f_out[...] = (jnp.dot(species[...], ea_W[...],
                          preferred_element_type=jnp.float32) + ea_b[...])
    proj = (2.0 * jnp.pi) * (t[...] * rff_W[...])
    feats = jnp.concatenate([jnp.cos(proj), jnp.sin(proj)], axis=-1)
    u = _silu(jnp.dot(feats, tm_W1[...], preferred_element_type=jnp.float32)
              + tm_b1[...])
    te_out[...] = (jnp.dot(u, tm_W2[...], preferred_element_type=jnp.float32)
                   + tm_b2[...])


def _nodes_pre(species, t, ea_W, ea_b, rff_W, tm_W1, tm_b1, tm_W2, tm_b2):
    nb = N // BN
    full = lambda shape: pl.BlockSpec(shape, lambda n: (0,) * len(shape))
    return pl.pallas_call(
        _nodes_pre_body,
        grid=(nb,),
        in_specs=[
            pl.BlockSpec((BN, 100), lambda n: (n, 0)),
            pl.BlockSpec((BN, 1), lambda n: (n, 0)),
            full((100, D)), full((1, D)), full((1, D // 2)),
            full((D, D)), full((1, D)), full((D, D)), full((1, D)),
        ],
        out_specs=[
            pl.BlockSpec((BN, D), lambda n: (n, 0)),
            pl.BlockSpec((BN, D), lambda n: (n, 0)),
        ],
        out_shape=[
            jax.ShapeDtypeStruct((N, D), jnp.float32),
            jax.ShapeDtypeStruct((N, D), jnp.float32),
        ],
    )(species, t, ea_W, ea_b, rff_W, tm_W1, tm_b1, tm_W2, tm_b2)


def _ea_body(edge_attr, W1, b1, W2, b2, ea_out):
    u = _silu(jnp.dot(edge_attr[...], W1[...],
                      preferred_element_type=jnp.float32) + b1[...])
    ea_out[...] = (jnp.dot(u, W2[...], preferred_element_type=jnp.float32)
                   + b2[...])


def _ea_mlp(edge_attr, W1, b1, W2, b2):
    nb = E // BE
    full = lambda shape: pl.BlockSpec(shape, lambda n: (0,) * len(shape))
    return pl.pallas_call(
        _ea_body,
        grid=(nb,),
        in_specs=[
            pl.BlockSpec((BE, 120), lambda n: (n, 0)),
            full((120, D)), full((1, D)), full((D, D)), full((1, D)),
        ],
        out_specs=pl.BlockSpec((BE, D), lambda n: (n, 0)),
        out_shape=jax.ShapeDtypeStruct((E, D), jnp.float32),
    )(edge_attr, W1, b1, W2, b2)


BE3 = 2048


def _edges_body(fi, fj, ea, ev, ps_W1, ps_b1, ps_W2, ps_b2,
                pv_W1, pv_b1, pv_W2, pv_b2, out):
    fi_v = fi[...]
    fj_v = fj[...]
    ea_v = ea[...]

    def mlp3(W1, b1, W2, b2):
        u = (jnp.dot(fi_v, W1[0:D], preferred_element_type=jnp.float32)
             + jnp.dot(fj_v, W1[D:2 * D], preferred_element_type=jnp.float32)
             + jnp.dot(ea_v, W1[2 * D:3 * D], preferred_element_type=jnp.float32)
             + b1[...])
        return jnp.dot(_silu(u), W2[...],
                       preferred_element_type=jnp.float32) + b2[...]

    rows = (pl.program_id(0) * BE3
            + lax.broadcasted_iota(jnp.int32, (BE3, 1), 0))
    valid = rows < E
    msg = mlp3(ps_W1, ps_b1, ps_W2, ps_b2) * fi_v
    vw = mlp3(pv_W1, pv_b1, pv_W2, pv_b2)
    ev_v = ev[...]
    out[0] = jnp.where(valid, msg, 0.0)
    out[1] = jnp.where(valid, vw * ev_v[:, 0:1], 0.0)
    out[2] = jnp.where(valid, vw * ev_v[:, 1:2], 0.0)
    out[3] = jnp.where(valid, vw * ev_v[:, 2:3], 0.0)


def _edges_mlp(fi, fj, ea, edge_vec, ps_W1, ps_b1, ps_W2, ps_b2,
               pv_W1, pv_b1, pv_W2, pv_b2):
    nb = EPAD // BE3
    full = lambda shape: pl.BlockSpec(shape, lambda n: (0,) * len(shape))
    return pl.pallas_call(
        _edges_body,
        grid=(nb,),
        in_specs=[
            pl.BlockSpec((BE3, D), lambda n: (n, 0)),
            pl.BlockSpec((BE3, D), lambda n: (n, 0)),
            pl.BlockSpec((BE3, D), lambda n: (jnp.minimum(n, 78), 0)),
            pl.BlockSpec((BE3, 3), lambda n: (jnp.minimum(n, 78), 0)),
            full((3 * D, D)), full((1, D)), full((D, D)), full((1, D)),
            full((3 * D, D)), full((1, D)), full((D, D)), full((1, D)),
        ],
        out_specs=pl.BlockSpec((4, BE3, D), lambda n: (0, n, 0)),
        out_shape=jax.ShapeDtypeStruct((4, EPAD, D), jnp.float32),
    )(fi, fj, ea, edge_vec, ps_W1, ps_b1, ps_W2, ps_b2,
      pv_W1, pv_b1, pv_W2, pv_b2)


def _h0_body(f, agg, te, W1, b1, W2, b2, h0_out):
    u = (jnp.dot(f[...], W1[0:D], preferred_element_type=jnp.float32)
         + jnp.dot(agg[...], W1[D:2 * D], preferred_element_type=jnp.float32)
         + b1[...])
    h0_out[...] = (jnp.dot(_silu(u), W2[...],
                           preferred_element_type=jnp.float32)
                   + b2[...] + te[...])


def _h0_mlp(f, agg, te, W1, b1, W2, b2):
    nb = N // BN
    full = lambda shape: pl.BlockSpec(shape, lambda n: (0,) * len(shape))
    return pl.pallas_call(
        _h0_body,
        grid=(nb,),
        in_specs=[
            pl.BlockSpec((BN, D), lambda n: (n, 0)),
            pl.BlockSpec((BN, D), lambda n: (n, 0)),
            pl.BlockSpec((BN, D), lambda n: (n, 0)),
            full((2 * D, D)), full((1, D)), full((D, D)), full((1, D)),
        ],
        out_specs=pl.BlockSpec((BN, D), lambda n: (n, 0)),
        out_shape=jax.ShapeDtypeStruct((N, D), jnp.float32),
    )(f, agg, te, W1, b1, W2, b2)


def _sc_gather(f, ig, jg):
    mesh = plsc.VectorSubcoreMesh(core_axis_name="c", subcore_axis_name="s")

    @functools.partial(
        pl.kernel,
        mesh=mesh,
        out_type=[jax.ShapeDtypeStruct((EPAD, D), jnp.float32),
                  jax.ShapeDtypeStruct((EPAD, D), jnp.float32)],
        scratch_types=[
            pltpu.VMEM((GPW, G), jnp.int32),
            pltpu.VMEM((GPW, G), jnp.int32),
            pltpu.VMEM((G, D), jnp.float32),
            pltpu.VMEM((G, D), jnp.float32),
            pltpu.VMEM((G, D), jnp.float32),
            pltpu.VMEM((G, D), jnp.float32),
            pltpu.SemaphoreType.DMA,
            pltpu.SemaphoreType.DMA,
            pltpu.SemaphoreType.DMA,
        ],
    )
    def k(f_hbm, ig_hbm, jg_hbm, fi_hbm, fj_hbm,
          iv, jv, riA, rjA, riB, rjB, gsA, gsB, ws):
        c = lax.axis_index("c")
        s = lax.axis_index("s")
        wid = s * 2 + c
        start = wid * GPW
        pltpu.sync_copy(ig_hbm.at[pl.ds(start, GPW)], iv)
        pltpu.sync_copy(jg_hbm.at[pl.ds(start, GPW)], jv)

        def body(t, carry):
            p = 2 * t
            q = 2 * t + 1
            cA1 = pltpu.async_copy(f_hbm.at[iv.at[p]], riA, gsA)
            cA2 = pltpu.async_copy(f_hbm.at[jv.at[p]], rjA, gsA)
            cB1 = pltpu.async_copy(f_hbm.at[iv.at[q]], riB, gsB)
            cB2 = pltpu.async_copy(f_hbm.at[jv.at[q]], rjB, gsB)
            cA1.wait()
            cA2.wait()
            w1 = pltpu.async_copy(riA, fi_hbm.at[pl.ds((start + p) * G, G)],
                                  ws)
            w2 = pltpu.async_copy(rjA, fj_hbm.at[pl.ds((start + p) * G, G)],
                                  ws)
            cB1.wait()
            cB2.wait()
            w3 = pltpu.async_copy(riB, fi_hbm.at[pl.ds((start + q) * G, G)],
                                  ws)
            w4 = pltpu.async_copy(rjB, fj_hbm.at[pl.ds((start + q) * G, G)],
                                  ws)
            w1.wait()
            w2.wait()
            w3.wait()
            w4.wait()
            return carry

        lax.fori_loop(0, GPW // 2, body, 0)

    return k(f, ig, jg)


NSUB = 16
GPSU = GPAD // NSUB
NPAD = 10240
NROWS = NPAD // NSUB
ZR = 128


def _sc_scatter(stacked, jg):
    mesh = plsc.VectorSubcoreMesh(core_axis_name="c", subcore_axis_name="s")

    @functools.partial(
        pl.kernel,
        mesh=mesh,
        out_type=jax.ShapeDtypeStruct((4, NPAD, D), jnp.float32),
        scratch_types=[
            pltpu.VMEM((GPSU, G), jnp.int32),
            pltpu.VMEM((G, D), jnp.float32),
            pltpu.VMEM((G, D), jnp.float32),
            pltpu.VMEM_SHARED((NPAD, D), jnp.float32),
            pltpu.SemaphoreType.DMA,
            pltpu.SemaphoreType.DMA,
        ],
    )
    def k(st_hbm, jg_hbm, out_hbm, idxv, dbufA, dbufB, acc, lsA, lsB):
        c = lax.axis_index("c")
        s = lax.axis_index("s")

        start = s * GPSU
        pltpu.sync_copy(jg_hbm.at[pl.ds(start, GPSU)], idxv)
        NIT = GPSU // 2

        for p in range(2):
            pass_idx = c * 2 + p

            def zb(tt, carry):
                dbufA[tt // 8, pl.ds((tt % 8) * 16, 16)] = jnp.zeros(
                    (16,), jnp.float32)
                return carry

            lax.fori_loop(0, ZR * 8, zb, 0)
            for q in range(NROWS // ZR):
                pltpu.sync_copy(dbufA,
                                acc.at[pl.ds(s * NROWS + q * ZR, ZR)])
            plsc.subcore_barrier()

            def body(t, carry):
                rp = 2 * t
                rq = 2 * t + 1
                lA = pltpu.async_copy(
                    st_hbm.at[pass_idx, pl.ds((start + rp) * G, G)],
                    dbufA, lsA)
                lB = pltpu.async_copy(
                    st_hbm.at[pass_idx, pl.ds((start + rq) * G, G)],
                    dbufB, lsB)
                lA.wait()
                pltpu.sync_copy(dbufA, acc.at[idxv.at[rp]], add=True)
                lB.wait()
                pltpu.sync_copy(dbufB, acc.at[idxv.at[rq]], add=True)
                return carry

            lax.fori_loop(0, NIT, body, 0)
            plsc.subcore_barrier()
            pltpu.sync_copy(acc.at[pl.ds(s * NROWS, NROWS)],
                            out_hbm.at[pass_idx, pl.ds(s * NROWS, NROWS)])
            plsc.subcore_barrier()

    return k(stacked, jg)


def kernel(species, edge_index, edge_attr, edge_vec, t,
           ea_W, ea_b, eb_W1, eb_b1, eb_W2, eb_b2,
           ps_W1, ps_b1, ps_W2, ps_b2,
           ph_W1, ph_b1, ph_W2, ph_b2,
           pv_W1, pv_b1, pv_W2, pv_b2,
           rff_W, tm_W1, tm_b1, tm_W2, tm_b2):
    r = lambda b: b.reshape(1, -1)
    i = edge_index[0]
    j = edge_index[1]

    f, te = _nodes_pre(species, t, ea_W, r(ea_b), rff_W,
                       tm_W1, r(tm_b1), tm_W2, r(tm_b2))
    ea = _ea_mlp(edge_attr, eb_W1, r(eb_b1), eb_W2, r(eb_b2))

    pad = EPAD - E
    ig = jnp.pad(i, (0, pad)).reshape(GPAD, G)
    jg = jnp.pad(j, (0, pad)).reshape(GPAD, G)
    fi, fj = _sc_gather(f, ig, jg)

    stacked = _edges_mlp(fi, fj, ea, edge_vec,
                         ps_W1, r(ps_b1), ps_W2, r(ps_b2),
                         pv_W1, r(pv_b1), pv_W2, r(pv_b2))

    segs = _sc_scatter(stacked, jg)
    agg = segs[0]
    v0 = jnp.transpose(segs[1:4, :N], (1, 2, 0))

    h0 = _h0_mlp(f, agg, te, ph_W1, r(ph_b1), ph_W2, r(ph_b2))
    return (h0, v0, ea)


# speedup vs baseline: 16.0877x; 1.0401x over previous
import functools

import jax
import jax.numpy as jnp
from jax import lax
from jax.experimental import pallas as pl
from jax.experimental.pallas import tpu as pltpu
from jax.experimental.pallas import tpu_sc as plsc

N = 10000
E = 160000
D = 128

BN = 2000
BE = 2000

G = 128
NW = 32
NGROUPS = E // G
GPAD = ((NGROUPS + NW - 1) // NW) * NW
EPAD = GPAD * G
GPW = GPAD // NW


def _silu(x):
    return x * jax.nn.sigmoid(x)


def _nodes_pre_body(species, t, ea_W, ea_b, rff_W, tm_W1, tm_b1, tm_W2, tm_b2,
                    f_out, te_out):
    f_out[...] = (jnp.dot(species[...], ea_W[...],
                          preferred_element_type=jnp.float32) + ea_b[...])
    proj = (2.0 * jnp.pi) * (t[...] * rff_W[...])
    feats = jnp.concatenate([jnp.cos(proj), jnp.sin(proj)], axis=-1)
    u = _silu(jnp.dot(feats, tm_W1[...], preferred_element_type=jnp.float32)
              + tm_b1[...])
    te_out[...] = (jnp.dot(u, tm_W2[...], preferred_element_type=jnp.float32)
                   + tm_b2[...])


def _nodes_pre(species, t, ea_W, ea_b, rff_W, tm_W1, tm_b1, tm_W2, tm_b2):
    nb = N // BN
    full = lambda shape: pl.BlockSpec(shape, lambda n: (0,) * len(shape))
    return pl.pallas_call(
        _nodes_pre_body,
        grid=(nb,),
        in_specs=[
            pl.BlockSpec((BN, 100), lambda n: (n, 0)),
            pl.BlockSpec((BN, 1), lambda n: (n, 0)),
            full((100, D)), full((1, D)), full((1, D // 2)),
            full((D, D)), full((1, D)), full((D, D)), full((1, D)),
        ],
        out_specs=[
            pl.BlockSpec((BN, D), lambda n: (n, 0)),
            pl.BlockSpec((BN, D), lambda n: (n, 0)),
        ],
        out_shape=[
            jax.ShapeDtypeStruct((N, D), jnp.float32),
            jax.ShapeDtypeStruct((N, D), jnp.float32),
        ],
    )(species, t, ea_W, ea_b, rff_W, tm_W1, tm_b1, tm_W2, tm_b2)


def _ea_body(edge_attr, W1, b1, W2, b2, ea_out):
    u = _silu(jnp.dot(edge_attr[...], W1[...],
                      preferred_element_type=jnp.float32) + b1[...])
    ea_out[...] = (jnp.dot(u, W2[...], preferred_element_type=jnp.float32)
                   + b2[...])


def _ea_mlp(edge_attr, W1, b1, W2, b2):
    nb = E // BE
    full = lambda shape: pl.BlockSpec(shape, lambda n: (0,) * len(shape))
    return pl.pallas_call(
        _ea_body,
        grid=(nb,),
        in_specs=[
            pl.BlockSpec((BE, 120), lambda n: (n, 0)),
            full((120, D)), full((1, D)), full((D, D)), full((1, D)),
        ],
        out_specs=pl.BlockSpec((BE, D), lambda n: (n, 0)),
        out_shape=jax.ShapeDtypeStruct((E, D), jnp.float32),
    )(edge_attr, W1, b1, W2, b2)


BE3 = 2048


def _edges_body(fi, fj, ea, ev, ps_W1, ps_b1, ps_W2, ps_b2,
                pv_W1, pv_b1, pv_W2, pv_b2, out):
    fi_v = fi[...]
    fj_v = fj[...]
    ea_v = ea[...]

    def mlp3(W1, b1, W2, b2):
        u = (jnp.dot(fi_v, W1[0:D], preferred_element_type=jnp.float32)
             + jnp.dot(fj_v, W1[D:2 * D], preferred_element_type=jnp.float32)
             + jnp.dot(ea_v, W1[2 * D:3 * D], preferred_element_type=jnp.float32)
             + b1[...])
        return jnp.dot(_silu(u), W2[...],
                       preferred_element_type=jnp.float32) + b2[...]

    rows = (pl.program_id(0) * BE3
            + lax.broadcasted_iota(jnp.int32, (BE3, 1), 0))
    valid = rows < E
    msg = mlp3(ps_W1, ps_b1, ps_W2, ps_b2) * fi_v
    vw = mlp3(pv_W1, pv_b1, pv_W2, pv_b2)
    ev_v = ev[...]
    out[0] = jnp.where(valid, msg, 0.0)
    out[1] = jnp.where(valid, vw * ev_v[:, 0:1], 0.0)
    out[2] = jnp.where(valid, vw * ev_v[:, 1:2], 0.0)
    out[3] = jnp.where(valid, vw * ev_v[:, 2:3], 0.0)


def _edges_mlp(fi, fj, ea, edge_vec, ps_W1, ps_b1, ps_W2, ps_b2,
               pv_W1, pv_b1, pv_W2, pv_b2):
    nb = EPAD // BE3
    full = lambda shape: pl.BlockSpec(shape, lambda n: (0,) * len(shape))
    return pl.pallas_call(
        _edges_body,
        grid=(nb,),
        in_specs=[
            pl.BlockSpec((BE3, D), lambda n: (n, 0)),
            pl.BlockSpec((BE3, D), lambda n: (n, 0)),
            pl.BlockSpec((BE3, D), lambda n: (jnp.minimum(n, 78), 0)),
            pl.BlockSpec((BE3, 3), lambda n: (jnp.minimum(n, 78), 0)),
            full((3 * D, D)), full((1, D)), full((D, D)), full((1, D)),
            full((3 * D, D)), full((1, D)), full((D, D)), full((1, D)),
        ],
        out_specs=pl.BlockSpec((4, BE3, D), lambda n: (0, n, 0)),
        out_shape=jax.ShapeDtypeStruct((4, EPAD, D), jnp.float32),
    )(fi, fj, ea, edge_vec, ps_W1, ps_b1, ps_W2, ps_b2,
      pv_W1, pv_b1, pv_W2, pv_b2)


def _h0_body(f, agg, te, W1, b1, W2, b2, h0_out):
    u = (jnp.dot(f[...], W1[0:D], preferred_element_type=jnp.float32)
         + jnp.dot(agg[...], W1[D:2 * D], preferred_element_type=jnp.float32)
         + b1[...])
    h0_out[...] = (jnp.dot(_silu(u), W2[...],
                           preferred_element_type=jnp.float32)
                   + b2[...] + te[...])


def _h0_mlp(f, agg, te, W1, b1, W2, b2):
    nb = N // BN
    full = lambda shape: pl.BlockSpec(shape, lambda n: (0,) * len(shape))
    return pl.pallas_call(
        _h0_body,
        grid=(nb,),
        in_specs=[
            pl.BlockSpec((BN, D), lambda n: (n, 0)),
            pl.BlockSpec((BN, D), lambda n: (n, 0)),
            pl.BlockSpec((BN, D), lambda n: (n, 0)),
            full((2 * D, D)), full((1, D)), full((D, D)), full((1, D)),
        ],
        out_specs=pl.BlockSpec((BN, D), lambda n: (n, 0)),
        out_shape=jax.ShapeDtypeStruct((N, D), jnp.float32),
    )(f, agg, te, W1, b1, W2, b2)


def _sc_gather(f, ig, jg):
    mesh = plsc.VectorSubcoreMesh(core_axis_name="c", subcore_axis_name="s")

    @functools.partial(
        pl.kernel,
        mesh=mesh,
        out_type=[jax.ShapeDtypeStruct((EPAD, D), jnp.float32),
                  jax.ShapeDtypeStruct((EPAD, D), jnp.float32)],
        scratch_types=[
            pltpu.VMEM((GPW, G), jnp.int32),
            pltpu.VMEM((GPW, G), jnp.int32),
            pltpu.VMEM((G, D), jnp.float32),
            pltpu.VMEM((G, D), jnp.float32),
            pltpu.VMEM((G, D), jnp.float32),
            pltpu.VMEM((G, D), jnp.float32),
            pltpu.SemaphoreType.DMA,
            pltpu.SemaphoreType.DMA,
            pltpu.SemaphoreType.DMA,
        ],
    )
    def k(f_hbm, ig_hbm, jg_hbm, fi_hbm, fj_hbm,
          iv, jv, riA, rjA, riB, rjB, gsA, gsB, ws):
        c = lax.axis_index("c")
        s = lax.axis_index("s")
        wid = s * 2 + c
        start = wid * GPW
        pltpu.sync_copy(ig_hbm.at[pl.ds(start, GPW)], iv)
        pltpu.sync_copy(jg_hbm.at[pl.ds(start, GPW)], jv)

        def drain_wb():
            pltpu.make_async_copy(riA, fi_hbm.at[pl.ds(0, G)], ws).wait()
            pltpu.make_async_copy(rjA, fj_hbm.at[pl.ds(0, G)], ws).wait()
            pltpu.make_async_copy(riB, fi_hbm.at[pl.ds(0, G)], ws).wait()
            pltpu.make_async_copy(rjB, fj_hbm.at[pl.ds(0, G)], ws).wait()

        def body(t, carry):
            p = 2 * t
            q = 2 * t + 1

            @pl.when(t > 0)
            def _():
                drain_wb()

            cA1 = pltpu.async_copy(f_hbm.at[iv.at[p]], riA, gsA)
            cA2 = pltpu.async_copy(f_hbm.at[jv.at[p]], rjA, gsA)
            cB1 = pltpu.async_copy(f_hbm.at[iv.at[q]], riB, gsB)
            cB2 = pltpu.async_copy(f_hbm.at[jv.at[q]], rjB, gsB)
            cA1.wait()
            cA2.wait()
            pltpu.async_copy(riA, fi_hbm.at[pl.ds((start + p) * G, G)], ws)
            pltpu.async_copy(rjA, fj_hbm.at[pl.ds((start + p) * G, G)], ws)
            cB1.wait()
            cB2.wait()
            pltpu.async_copy(riB, fi_hbm.at[pl.ds((start + q) * G, G)], ws)
            pltpu.async_copy(rjB, fj_hbm.at[pl.ds((start + q) * G, G)], ws)
            return carry

        lax.fori_loop(0, GPW // 2, body, 0)
        drain_wb()

    return k(f, ig, jg)


NSUB = 16
GPSU = GPAD // NSUB
NPAD = 10240
NROWS = NPAD // NSUB
ZR = 128


def _sc_scatter(stacked, jg):
    mesh = plsc.VectorSubcoreMesh(core_axis_name="c", subcore_axis_name="s")

    @functools.partial(
        pl.kernel,
        mesh=mesh,
        out_type=jax.ShapeDtypeStruct((4, NPAD, D), jnp.float32),
        scratch_types=[
            pltpu.VMEM((GPSU, G), jnp.int32),
            pltpu.VMEM((G, D), jnp.float32),
            pltpu.VMEM((G, D), jnp.float32),
            pltpu.VMEM_SHARED((NPAD, D), jnp.float32),
            pltpu.SemaphoreType.DMA,
            pltpu.SemaphoreType.DMA,
        ],
    )
    def k(st_hbm, jg_hbm, out_hbm, idxv, dbufA, dbufB, acc, lsA, lsB):
        c = lax.axis_index("c")
        s = lax.axis_index("s")

        start = s * GPSU
        pltpu.sync_copy(jg_hbm.at[pl.ds(start, GPSU)], idxv)
        NIT = GPSU // 2

        for p in range(2):
            pass_idx = c * 2 + p

            def zb(tt, carry):
                dbufA[tt // 8, pl.ds((tt % 8) * 16, 16)] = jnp.zeros(
                    (16,), jnp.float32)
                return carry

            lax.fori_loop(0, ZR * 8, zb, 0)
            for q in range(NROWS // ZR):
                pltpu.sync_copy(dbufA,
                                acc.at[pl.ds(s * NROWS + q * ZR, ZR)])
            plsc.subcore_barrier()

            pltpu.async_copy(st_hbm.at[pass_idx, pl.ds(start * G, G)],
                             dbufA, lsA)

            def body(t, carry):
                rp = 2 * t
                rq = 2 * t + 1
                pltpu.make_async_copy(st_hbm.at[0, pl.ds(0, G)],
                                      dbufA, lsA).wait()
                pltpu.async_copy(
                    st_hbm.at[pass_idx, pl.ds((start + rq) * G, G)],
                    dbufB, lsB)
                pltpu.sync_copy(dbufA, acc.at[idxv.at[rp]], add=True)
                pltpu.make_async_copy(st_hbm.at[0, pl.ds(0, G)],
                                      dbufB, lsB).wait()

                @pl.when(t < NIT - 1)
                def _():
                    pltpu.async_copy(
                        st_hbm.at[pass_idx, pl.ds((start + rp + 2) * G, G)],
                        dbufA, lsA)

                pltpu.sync_copy(dbufB, acc.at[idxv.at[rq]], add=True)
                return carry

            lax.fori_loop(0, NIT, body, 0)
            plsc.subcore_barrier()
            pltpu.sync_copy(acc.at[pl.ds(s * NROWS, NROWS)],
                            out_hbm.at[pass_idx, pl.ds(s * NROWS, NROWS)])
            plsc.subcore_barrier()

    return k(stacked, jg)


def kernel(species, edge_index, edge_attr, edge_vec, t,
           ea_W, ea_b, eb_W1, eb_b1, eb_W2, eb_b2,
           ps_W1, ps_b1, ps_W2, ps_b2,
           ph_W1, ph_b1, ph_W2, ph_b2,
           pv_W1, pv_b1, pv_W2, pv_b2,
           rff_W, tm_W1, tm_b1, tm_W2, tm_b2):
    r = lambda b: b.reshape(1, -1)
    i = edge_index[0]
    j = edge_index[1]

    f, te = _nodes_pre(species, t, ea_W, r(ea_b), rff_W,
                       tm_W1, r(tm_b1), tm_W2, r(tm_b2))
    ea = _ea_mlp(edge_attr, eb_W1, r(eb_b1), eb_W2, r(eb_b2))

    pad = EPAD - E
    ig = jnp.pad(i, (0, pad)).reshape(GPAD, G)
    jg = jnp.pad(j, (0, pad)).reshape(GPAD, G)
    fi, fj = _sc_gather(f, ig, jg)

    stacked = _edges_mlp(fi, fj, ea, edge_vec,
                         ps_W1, r(ps_b1), ps_W2, r(ps_b2),
                         pv_W1, r(pv_b1), pv_W2, r(pv_b2))

    segs = _sc_scatter(stacked, jg)
    agg = segs[0]
    v0 = jnp.transpose(segs[1:4, :N], (1, 2, 0))

    h0 = _h0_mlp(f, agg, te, ph_W1, r(ph_b1), ph_W2, r(ph_b2))
    return (h0, v0, ea)


# speedup vs baseline: 22.7526x; 1.4143x over previous
import functools

import jax
import jax.numpy as jnp
from jax import lax
from jax.experimental import pallas as pl
from jax.experimental.pallas import tpu as pltpu
from jax.experimental.pallas import tpu_sc as plsc

N = 10000
E = 160000
D = 128

BN = 2000
BE = 2000

G = 128
NW = 32
NGROUPS = E // G
GPAD = ((NGROUPS + NW - 1) // NW) * NW
EPAD = GPAD * G
GPW = GPAD // NW


def _silu(x):
    return x * jax.nn.sigmoid(x)


def _nodes_pre_body(species, t, ea_W, ea_b, rff_W, tm_W1, tm_b1, tm_W2, tm_b2,
                    f_out, te_out):
    f_out[...] = (jnp.dot(species[...], ea_W[...],
                          preferred_element_type=jnp.float32) + ea_b[...])
    proj = (2.0 * jnp.pi) * (t[...] * rff_W[...])
    feats = jnp.concatenate([jnp.cos(proj), jnp.sin(proj)], axis=-1)
    u = _silu(jnp.dot(feats, tm_W1[...], preferred_element_type=jnp.float32)
              + tm_b1[...])
    te_out[...] = (jnp.dot(u, tm_W2[...], preferred_element_type=jnp.float32)
                   + tm_b2[...])


def _nodes_pre(species, t, ea_W, ea_b, rff_W, tm_W1, tm_b1, tm_W2, tm_b2):
    nb = N // BN
    full = lambda shape: pl.BlockSpec(shape, lambda n: (0,) * len(shape))
    return pl.pallas_call(
        _nodes_pre_body,
        grid=(nb,),
        in_specs=[
            pl.BlockSpec((BN, 100), lambda n: (n, 0)),
            pl.BlockSpec((BN, 1), lambda n: (n, 0)),
            full((100, D)), full((1, D)), full((1, D // 2)),
            full((D, D)), full((1, D)), full((D, D)), full((1, D)),
        ],
        out_specs=[
            pl.BlockSpec((BN, D), lambda n: (n, 0)),
            pl.BlockSpec((BN, D), lambda n: (n, 0)),
        ],
        out_shape=[
            jax.ShapeDtypeStruct((NPAD, D), jnp.float32),
            jax.ShapeDtypeStruct((NPAD, D), jnp.float32),
        ],
    )(species, t, ea_W, ea_b, rff_W, tm_W1, tm_b1, tm_W2, tm_b2)


def _ea_body(edge_attr, W1, b1, W2, b2, ea_out):
    u = _silu(jnp.dot(edge_attr[...], W1[...],
                      preferred_element_type=jnp.float32) + b1[...])
    ea_out[...] = (jnp.dot(u, W2[...], preferred_element_type=jnp.float32)
                   + b2[...])


def _ea_mlp(edge_attr, W1, b1, W2, b2):
    nb = E // BE
    full = lambda shape: pl.BlockSpec(shape, lambda n: (0,) * len(shape))
    return pl.pallas_call(
        _ea_body,
        grid=(nb,),
        in_specs=[
            pl.BlockSpec((BE, 120), lambda n: (n, 0)),
            full((120, D)), full((1, D)), full((D, D)), full((1, D)),
        ],
        out_specs=pl.BlockSpec((BE, D), lambda n: (n, 0)),
        out_shape=jax.ShapeDtypeStruct((E, D), jnp.float32),
    )(edge_attr, W1, b1, W2, b2)


BE3 = 2048


def _edges_body(fi, fj, ea, ev, ps_W1, ps_b1, ps_W2, ps_b2,
                pv_W1, pv_b1, pv_W2, pv_b2, out):
    fi_v = fi[...]
    fj_v = fj[...]
    ea_v = ea[...]

    def mlp3(W1, b1, W2, b2):
        u = (jnp.dot(fi_v, W1[0:D], preferred_element_type=jnp.float32)
             + jnp.dot(fj_v, W1[D:2 * D], preferred_element_type=jnp.float32)
             + jnp.dot(ea_v, W1[2 * D:3 * D], preferred_element_type=jnp.float32)
             + b1[...])
        return jnp.dot(_silu(u), W2[...],
                       preferred_element_type=jnp.float32) + b2[...]

    rows = (pl.program_id(0) * BE3
            + lax.broadcasted_iota(jnp.int32, (BE3, 1), 0))
    valid = rows < E
    msg = mlp3(ps_W1, ps_b1, ps_W2, ps_b2) * fi_v
    vw = mlp3(pv_W1, pv_b1, pv_W2, pv_b2)
    ev_v = ev[...]
    out[0] = jnp.where(valid, msg, 0.0)
    out[1] = jnp.where(valid, vw * ev_v[:, 0:1], 0.0)
    out[2] = jnp.where(valid, vw * ev_v[:, 1:2], 0.0)
    out[3] = jnp.where(valid, vw * ev_v[:, 2:3], 0.0)


def _edges_mlp(fi, fj, ea, edge_vec, ps_W1, ps_b1, ps_W2, ps_b2,
               pv_W1, pv_b1, pv_W2, pv_b2):
    nb = EPAD // BE3
    full = lambda shape: pl.BlockSpec(shape, lambda n: (0,) * len(shape))
    return pl.pallas_call(
        _edges_body,
        grid=(nb,),
        in_specs=[
            pl.BlockSpec((BE3, D), lambda n: (n, 0)),
            pl.BlockSpec((BE3, D), lambda n: (n, 0)),
            pl.BlockSpec((BE3, D), lambda n: (jnp.minimum(n, 78), 0)),
            pl.BlockSpec((BE3, 3), lambda n: (jnp.minimum(n, 78), 0)),
            full((3 * D, D)), full((1, D)), full((D, D)), full((1, D)),
            full((3 * D, D)), full((1, D)), full((D, D)), full((1, D)),
        ],
        out_specs=pl.BlockSpec((4, BE3, D), lambda n: (0, n, 0)),
        out_shape=jax.ShapeDtypeStruct((4, EPAD, D), jnp.float32),
    )(fi, fj, ea, edge_vec, ps_W1, ps_b1, ps_W2, ps_b2,
      pv_W1, pv_b1, pv_W2, pv_b2)


def _h0_body(f, agg, te, W1, b1, W2, b2, h0_out):
    u = (jnp.dot(f[...], W1[0:D], preferred_element_type=jnp.float32)
         + jnp.dot(agg[...], W1[D:2 * D], preferred_element_type=jnp.float32)
         + b1[...])
    h0_out[...] = (jnp.dot(_silu(u), W2[...],
                           preferred_element_type=jnp.float32)
                   + b2[...] + te[...])


def _h0_mlp(f, agg, te, W1, b1, W2, b2):
    nb = N // BN
    full = lambda shape: pl.BlockSpec(shape, lambda n: (0,) * len(shape))
    return pl.pallas_call(
        _h0_body,
        grid=(nb,),
        in_specs=[
            pl.BlockSpec((BN, D), lambda n: (n, 0)),
            pl.BlockSpec((BN, D), lambda n: (n, 0)),
            pl.BlockSpec((BN, D), lambda n: (n, 0)),
            full((2 * D, D)), full((1, D)), full((D, D)), full((1, D)),
        ],
        out_specs=pl.BlockSpec((BN, D), lambda n: (n, 0)),
        out_shape=jax.ShapeDtypeStruct((N, D), jnp.float32),
    )(f, agg, te, W1, b1, W2, b2)


def _sc_gather(f, ig, jg):
    mesh = plsc.VectorSubcoreMesh(core_axis_name="c", subcore_axis_name="s")

    @functools.partial(
        pl.kernel,
        mesh=mesh,
        out_type=[jax.ShapeDtypeStruct((EPAD, D), jnp.float32),
                  jax.ShapeDtypeStruct((EPAD, D), jnp.float32)],
        scratch_types=[
            pltpu.VMEM_SHARED((NPAD, D), jnp.float32),
            pltpu.VMEM((GPW, G), jnp.int32),
            pltpu.VMEM((GPW, G), jnp.int32),
            pltpu.VMEM((G, D), jnp.float32),
            pltpu.VMEM((G, D), jnp.float32),
            pltpu.SemaphoreType.DMA,
            pltpu.SemaphoreType.DMA,
        ],
    )
    def k(f_hbm, ig_hbm, jg_hbm, fi_hbm, fj_hbm,
          fsh, iv, jv, ri, rj, gs, ws):
        c = lax.axis_index("c")
        s = lax.axis_index("s")
        wid = s * 2 + c
        start = wid * GPW
        pltpu.sync_copy(f_hbm.at[pl.ds(s * NROWS, NROWS)],
                        fsh.at[pl.ds(s * NROWS, NROWS)])
        pltpu.sync_copy(ig_hbm.at[pl.ds(start, GPW)], iv)
        pltpu.sync_copy(jg_hbm.at[pl.ds(start, GPW)], jv)
        plsc.subcore_barrier()

        def drain_wb():
            pltpu.make_async_copy(ri, fi_hbm.at[pl.ds(0, G)], ws).wait()
            pltpu.make_async_copy(rj, fj_hbm.at[pl.ds(0, G)], ws).wait()

        def body(t, carry):
            @pl.when(t > 0)
            def _():
                drain_wb()

            ci = pltpu.async_copy(fsh.at[iv.at[t]], ri, gs)
            cj = pltpu.async_copy(fsh.at[jv.at[t]], rj, gs)
            ci.wait()
            pltpu.async_copy(ri, fi_hbm.at[pl.ds((start + t) * G, G)], ws)
            cj.wait()
            pltpu.async_copy(rj, fj_hbm.at[pl.ds((start + t) * G, G)], ws)
            return carry

        lax.fori_loop(0, GPW, body, 0)
        drain_wb()

    return k(f, ig, jg)


NSUB = 16
GPSU = GPAD // NSUB
NPAD = 10240
NROWS = NPAD // NSUB
ZR = 128


def _sc_scatter(stacked, jg):
    mesh = plsc.VectorSubcoreMesh(core_axis_name="c", subcore_axis_name="s")

    @functools.partial(
        pl.kernel,
        mesh=mesh,
        out_type=jax.ShapeDtypeStruct((4, NPAD, D), jnp.float32),
        scratch_types=[
            pltpu.VMEM((GPSU, G), jnp.int32),
            pltpu.VMEM((G, D), jnp.float32),
            pltpu.VMEM((G, D), jnp.float32),
            pltpu.VMEM_SHARED((NPAD, D), jnp.float32),
            pltpu.SemaphoreType.DMA,
            pltpu.SemaphoreType.DMA,
        ],
    )
    def k(st_hbm, jg_hbm, out_hbm, idxv, dbufA, dbufB, acc, lsA, lsB):
        c = lax.axis_index("c")
        s = lax.axis_index("s")

        start = s * GPSU
        pltpu.sync_copy(jg_hbm.at[pl.ds(start, GPSU)], idxv)
        NIT = GPSU // 2

        for p in range(2):
            pass_idx = c * 2 + p

            def zb(tt, carry):
                dbufA[tt // 8, pl.ds((tt % 8) * 16, 16)] = jnp.zeros(
                    (16,), jnp.float32)
                return carry

            lax.fori_loop(0, ZR * 8, zb, 0)
            for q in range(NROWS // ZR):
                pltpu.sync_copy(dbufA,
                                acc.at[pl.ds(s * NROWS + q * ZR, ZR)])
            plsc.subcore_barrier()

            pltpu.async_copy(st_hbm.at[pass_idx, pl.ds(start * G, G)],
                             dbufA, lsA)

            def body(t, carry):
                rp = 2 * t
                rq = 2 * t + 1
                pltpu.make_async_copy(st_hbm.at[0, pl.ds(0, G)],
                                      dbufA, lsA).wait()
                pltpu.async_copy(
                    st_hbm.at[pass_idx, pl.ds((start + rq) * G, G)],
                    dbufB, lsB)
                pltpu.sync_copy(dbufA, acc.at[idxv.at[rp]], add=True)
                pltpu.make_async_copy(st_hbm.at[0, pl.ds(0, G)],
                                      dbufB, lsB).wait()

                @pl.when(t < NIT - 1)
                def _():
                    pltpu.async_copy(
                        st_hbm.at[pass_idx, pl.ds((start + rp + 2) * G, G)],
                        dbufA, lsA)

                pltpu.sync_copy(dbufB, acc.at[idxv.at[rq]], add=True)
                return carry

            lax.fori_loop(0, NIT, body, 0)
            plsc.subcore_barrier()
            pltpu.sync_copy(acc.at[pl.ds(s * NROWS, NROWS)],
                            out_hbm.at[pass_idx, pl.ds(s * NROWS, NROWS)])
            plsc.subcore_barrier()

    return k(stacked, jg)


def kernel(species, edge_index, edge_attr, edge_vec, t,
           ea_W, ea_b, eb_W1, eb_b1, eb_W2, eb_b2,
           ps_W1, ps_b1, ps_W2, ps_b2,
           ph_W1, ph_b1, ph_W2, ph_b2,
           pv_W1, pv_b1, pv_W2, pv_b2,
           rff_W, tm_W1, tm_b1, tm_W2, tm_b2):
    r = lambda b: b.reshape(1, -1)
    i = edge_index[0]
    j = edge_index[1]

    f, te = _nodes_pre(species, t, ea_W, r(ea_b), rff_W,
                       tm_W1, r(tm_b1), tm_W2, r(tm_b2))
    ea = _ea_mlp(edge_attr, eb_W1, r(eb_b1), eb_W2, r(eb_b2))

    pad = EPAD - E
    ig = jnp.pad(i, (0, pad)).reshape(GPAD, G)
    jg = jnp.pad(j, (0, pad)).reshape(GPAD, G)
    fi, fj = _sc_gather(f, ig, jg)

    stacked = _edges_mlp(fi, fj, ea, edge_vec,
                         ps_W1, r(ps_b1), ps_W2, r(ps_b2),
                         pv_W1, r(pv_b1), pv_W2, r(pv_b2))

    segs = _sc_scatter(stacked, jg)
    agg = segs[0]
    v0 = jnp.transpose(segs[1:4, :N], (1, 2, 0))

    h0 = _h0_mlp(f, agg, te, ph_W1, r(ph_b1), ph_W2, r(ph_b2))
    return (h0, v0, ea)


# speedup vs baseline: 24.9567x; 1.0969x over previous
import functools

import jax
import jax.numpy as jnp
from jax import lax
from jax.experimental import pallas as pl
from jax.experimental.pallas import tpu as pltpu
from jax.experimental.pallas import tpu_sc as plsc

N = 10000
E = 160000
D = 128

BN = 2000
BE = 2000

G = 128
NW = 32
NGROUPS = E // G
GPAD = ((NGROUPS + NW - 1) // NW) * NW
EPAD = GPAD * G
GPW = GPAD // NW


def _silu(x):
    return x * jax.nn.sigmoid(x)


def _nodes_pre_body(species, t, ea_W, ea_b, rff_W, tm_W1, tm_b1, tm_W2, tm_b2,
                    f_out, te_out):
    f_out[...] = (jnp.dot(species[...], ea_W[...],
                          preferred_element_type=jnp.float32) + ea_b[...])
    proj = (2.0 * jnp.pi) * (t[...] * rff_W[...])
    feats = jnp.concatenate([jnp.cos(proj), jnp.sin(proj)], axis=-1)
    u = _silu(jnp.dot(feats, tm_W1[...], preferred_element_type=jnp.float32)
              + tm_b1[...])
    te_out[...] = (jnp.dot(u, tm_W2[...], preferred_element_type=jnp.float32)
                   + tm_b2[...])


def _nodes_pre(species, t, ea_W, ea_b, rff_W, tm_W1, tm_b1, tm_W2, tm_b2):
    nb = N // BN
    full = lambda shape: pl.BlockSpec(shape, lambda n: (0,) * len(shape))
    return pl.pallas_call(
        _nodes_pre_body,
        grid=(nb,),
        in_specs=[
            pl.BlockSpec((BN, 100), lambda n: (n, 0)),
            pl.BlockSpec((BN, 1), lambda n: (n, 0)),
            full((100, D)), full((1, D)), full((1, D // 2)),
            full((D, D)), full((1, D)), full((D, D)), full((1, D)),
        ],
        out_specs=[
            pl.BlockSpec((BN, D), lambda n: (n, 0)),
            pl.BlockSpec((BN, D), lambda n: (n, 0)),
        ],
        out_shape=[
            jax.ShapeDtypeStruct((NPAD, D), jnp.float32),
            jax.ShapeDtypeStruct((NPAD, D), jnp.float32),
        ],
    )(species, t, ea_W, ea_b, rff_W, tm_W1, tm_b1, tm_W2, tm_b2)


def _ea_body(edge_attrT, W1, b1, W2, b2, ea_out):
    u = _silu(lax.dot_general(edge_attrT[...], W1[...],
                              (((0,), (0,)), ((), ())),
                              preferred_element_type=jnp.float32) + b1[...])
    ea_out[...] = (jnp.dot(u, W2[...], preferred_element_type=jnp.float32)
                   + b2[...])


def _ea_mlp(edge_attrT, W1, b1, W2, b2):
    be = 2048
    nb = (E + be - 1) // be
    full = lambda shape: pl.BlockSpec(shape, lambda n: (0,) * len(shape))
    return pl.pallas_call(
        _ea_body,
        grid=(nb,),
        in_specs=[
            pl.BlockSpec((120, be), lambda n: (0, n)),
            full((120, D)), full((1, D)), full((D, D)), full((1, D)),
        ],
        out_specs=pl.BlockSpec((be, D), lambda n: (n, 0)),
        out_shape=jax.ShapeDtypeStruct((E, D), jnp.float32),
    )(edge_attrT, W1, b1, W2, b2)


BE3 = 2048


def _edges_body(fi, fj, ea, ev, ps_W1, ps_b1, ps_W2, ps_b2,
                pv_W1, pv_b1, pv_W2, pv_b2, out):
    fi_v = fi[...]
    fj_v = fj[...]
    ea_v = ea[...]

    def mlp3(W1, b1, W2, b2):
        u = (jnp.dot(fi_v, W1[0:D], preferred_element_type=jnp.float32)
             + jnp.dot(fj_v, W1[D:2 * D], preferred_element_type=jnp.float32)
             + jnp.dot(ea_v, W1[2 * D:3 * D], preferred_element_type=jnp.float32)
             + b1[...])
        return jnp.dot(_silu(u), W2[...],
                       preferred_element_type=jnp.float32) + b2[...]

    rows = (pl.program_id(0) * BE3
            + lax.broadcasted_iota(jnp.int32, (BE3, 1), 0))
    valid = rows < E
    msg = mlp3(ps_W1, ps_b1, ps_W2, ps_b2) * fi_v
    vw = mlp3(pv_W1, pv_b1, pv_W2, pv_b2)
    ev_v = ev[...]
    out[0] = jnp.where(valid, msg, 0.0)
    out[1] = jnp.where(valid, vw * ev_v[:, 0:1], 0.0)
    out[2] = jnp.where(valid, vw * ev_v[:, 1:2], 0.0)
    out[3] = jnp.where(valid, vw * ev_v[:, 2:3], 0.0)


def _edges_mlp(fi, fj, ea, edge_vec, ps_W1, ps_b1, ps_W2, ps_b2,
               pv_W1, pv_b1, pv_W2, pv_b2):
    nb = EPAD // BE3
    full = lambda shape: pl.BlockSpec(shape, lambda n: (0,) * len(shape))
    return pl.pallas_call(
        _edges_body,
        grid=(nb,),
        in_specs=[
            pl.BlockSpec((BE3, D), lambda n: (n, 0)),
            pl.BlockSpec((BE3, D), lambda n: (n, 0)),
            pl.BlockSpec((BE3, D), lambda n: (jnp.minimum(n, 78), 0)),
            pl.BlockSpec((BE3, 3), lambda n: (jnp.minimum(n, 78), 0)),
            full((3 * D, D)), full((1, D)), full((D, D)), full((1, D)),
            full((3 * D, D)), full((1, D)), full((D, D)), full((1, D)),
        ],
        out_specs=pl.BlockSpec((4, BE3, D), lambda n: (0, n, 0)),
        out_shape=jax.ShapeDtypeStruct((4, EPAD, D), jnp.float32),
    )(fi, fj, ea, edge_vec, ps_W1, ps_b1, ps_W2, ps_b2,
      pv_W1, pv_b1, pv_W2, pv_b2)


def _h0_body(f, agg, te, W1, b1, W2, b2, h0_out):
    u = (jnp.dot(f[...], W1[0:D], preferred_element_type=jnp.float32)
         + jnp.dot(agg[...], W1[D:2 * D], preferred_element_type=jnp.float32)
         + b1[...])
    h0_out[...] = (jnp.dot(_silu(u), W2[...],
                           preferred_element_type=jnp.float32)
                   + b2[...] + te[...])


def _h0_mlp(f, agg, te, W1, b1, W2, b2):
    nb = N // BN
    full = lambda shape: pl.BlockSpec(shape, lambda n: (0,) * len(shape))
    return pl.pallas_call(
        _h0_body,
        grid=(nb,),
        in_specs=[
            pl.BlockSpec((BN, D), lambda n: (n, 0)),
            pl.BlockSpec((BN, D), lambda n: (n, 0)),
            pl.BlockSpec((BN, D), lambda n: (n, 0)),
            full((2 * D, D)), full((1, D)), full((D, D)), full((1, D)),
        ],
        out_specs=pl.BlockSpec((BN, D), lambda n: (n, 0)),
        out_shape=jax.ShapeDtypeStruct((N, D), jnp.float32),
    )(f, agg, te, W1, b1, W2, b2)


def _sc_gather(f, ig, jg):
    mesh = plsc.VectorSubcoreMesh(core_axis_name="c", subcore_axis_name="s")

    @functools.partial(
        pl.kernel,
        mesh=mesh,
        out_type=[jax.ShapeDtypeStruct((EPAD, D), jnp.float32),
                  jax.ShapeDtypeStruct((EPAD, D), jnp.float32)],
        scratch_types=[
            pltpu.VMEM_SHARED((NPAD, D), jnp.float32),
            pltpu.VMEM((GPW, G), jnp.int32),
            pltpu.VMEM((GPW, G), jnp.int32),
            pltpu.VMEM((G, D), jnp.float32),
            pltpu.VMEM((G, D), jnp.float32),
            pltpu.SemaphoreType.DMA,
            pltpu.SemaphoreType.DMA,
        ],
    )
    def k(f_hbm, ig_hbm, jg_hbm, fi_hbm, fj_hbm,
          fsh, iv, jv, ri, rj, gs, ws):
        c = lax.axis_index("c")
        s = lax.axis_index("s")
        wid = s * 2 + c
        start = wid * GPW
        pltpu.sync_copy(f_hbm.at[pl.ds(s * NROWS, NROWS)],
                        fsh.at[pl.ds(s * NROWS, NROWS)])
        pltpu.sync_copy(ig_hbm.at[pl.ds(start, GPW)], iv)
        pltpu.sync_copy(jg_hbm.at[pl.ds(start, GPW)], jv)
        plsc.subcore_barrier()

        def drain_wb():
            pltpu.make_async_copy(ri, fi_hbm.at[pl.ds(0, G)], ws).wait()
            pltpu.make_async_copy(rj, fj_hbm.at[pl.ds(0, G)], ws).wait()

        def body(t, carry):
            @pl.when(t > 0)
            def _():
                drain_wb()

            ci = pltpu.async_copy(fsh.at[iv.at[t]], ri, gs)
            cj = pltpu.async_copy(fsh.at[jv.at[t]], rj, gs)
            ci.wait()
            pltpu.async_copy(ri, fi_hbm.at[pl.ds((start + t) * G, G)], ws)
            cj.wait()
            pltpu.async_copy(rj, fj_hbm.at[pl.ds((start + t) * G, G)], ws)
            return carry

        lax.fori_loop(0, GPW, body, 0)
        drain_wb()

    return k(f, ig, jg)


NSUB = 16
GPSU = GPAD // NSUB
NPAD = 10240
NROWS = NPAD // NSUB
ZR = 128


def _sc_scatter(stacked, jg):
    mesh = plsc.VectorSubcoreMesh(core_axis_name="c", subcore_axis_name="s")

    @functools.partial(
        pl.kernel,
        mesh=mesh,
        out_type=jax.ShapeDtypeStruct((4, NPAD, D), jnp.float32),
        scratch_types=[
            pltpu.VMEM((GPSU, G), jnp.int32),
            pltpu.VMEM((G, D), jnp.float32),
            pltpu.VMEM((G, D), jnp.float32),
            pltpu.VMEM_SHARED((NPAD, D), jnp.float32),
            pltpu.SemaphoreType.DMA,
            pltpu.SemaphoreType.DMA,
        ],
    )
    def k(st_hbm, jg_hbm, out_hbm, idxv, dbufA, dbufB, acc, lsA, lsB):
        c = lax.axis_index("c")
        s = lax.axis_index("s")

        start = s * GPSU
        pltpu.sync_copy(jg_hbm.at[pl.ds(start, GPSU)], idxv)
        NIT = GPSU // 2

        for p in range(2):
            pass_idx = c * 2 + p

            def zb(tt, carry):
                dbufA[tt // 8, pl.ds((tt % 8) * 16, 16)] = jnp.zeros(
                    (16,), jnp.float32)
                return carry

            lax.fori_loop(0, ZR * 8, zb, 0)
            for q in range(NROWS // ZR):
                pltpu.sync_copy(dbufA,
                                acc.at[pl.ds(s * NROWS + q * ZR, ZR)])
            plsc.subcore_barrier()

            pltpu.async_copy(st_hbm.at[pass_idx, pl.ds(start * G, G)],
                             dbufA, lsA)

            def body(t, carry):
                rp = 2 * t
                rq = 2 * t + 1
                pltpu.make_async_copy(st_hbm.at[0, pl.ds(0, G)],
                                      dbufA, lsA).wait()
                pltpu.async_copy(
                    st_hbm.at[pass_idx, pl.ds((start + rq) * G, G)],
                    dbufB, lsB)
                pltpu.sync_copy(dbufA, acc.at[idxv.at[rp]], add=True)
                pltpu.make_async_copy(st_hbm.at[0, pl.ds(0, G)],
                                      dbufB, lsB).wait()

                @pl.when(t < NIT - 1)
                def _():
                    pltpu.async_copy(
                        st_hbm.at[pass_idx, pl.ds((start + rp + 2) * G, G)],
                        dbufA, lsA)

                pltpu.sync_copy(dbufB, acc.at[idxv.at[rq]], add=True)
                return carry

            lax.fori_loop(0, NIT, body, 0)
            plsc.subcore_barrier()
            pltpu.sync_copy(acc.at[pl.ds(s * NROWS, NROWS)],
                            out_hbm.at[pass_idx, pl.ds(s * NROWS, NROWS)])
            plsc.subcore_barrier()

    return k(stacked, jg)


def kernel(species, edge_index, edge_attr, edge_vec, t,
           ea_W, ea_b, eb_W1, eb_b1, eb_W2, eb_b2,
           ps_W1, ps_b1, ps_W2, ps_b2,
           ph_W1, ph_b1, ph_W2, ph_b2,
           pv_W1, pv_b1, pv_W2, pv_b2,
           rff_W, tm_W1, tm_b1, tm_W2, tm_b2):
    r = lambda b: b.reshape(1, -1)
    i = edge_index[0]
    j = edge_index[1]

    f, te = _nodes_pre(species, t, ea_W, r(ea_b), rff_W,
                       tm_W1, r(tm_b1), tm_W2, r(tm_b2))
    ea = _ea_mlp(edge_attr.T, eb_W1, r(eb_b1), eb_W2, r(eb_b2))

    pad = EPAD - E
    ig = jnp.pad(i, (0, pad)).reshape(GPAD, G)
    jg = jnp.pad(j, (0, pad)).reshape(GPAD, G)
    fi, fj = _sc_gather(f, ig, jg)

    stacked = _edges_mlp(fi, fj, ea, edge_vec,
                         ps_W1, r(ps_b1), ps_W2, r(ps_b2),
                         pv_W1, r(pv_b1), pv_W2, r(pv_b2))

    segs = _sc_scatter(stacked, jg)
    agg = segs[0]
    v0 = jnp.transpose(segs[1:4, :N], (1, 2, 0))

    h0 = _h0_mlp(f, agg, te, ph_W1, r(ph_b1), ph_W2, r(ph_b2))
    return (h0, v0, ea)


# speedup vs baseline: 24.9720x; 1.0006x over previous
import functools

import jax
import jax.numpy as jnp
from jax import lax
from jax.experimental import pallas as pl
from jax.experimental.pallas import tpu as pltpu
from jax.experimental.pallas import tpu_sc as plsc

N = 10000
E = 160000
D = 128

BN = 2000
BE = 2000

G = 128
NW = 32
NGROUPS = E // G
GPAD = ((NGROUPS + NW - 1) // NW) * NW
EPAD = GPAD * G
GPW = GPAD // NW


def _silu(x):
    return x * jax.nn.sigmoid(x)


def _nodes_pre_body(species, t, ea_W, ea_b, rff_W, tm_W1, tm_b1, tm_W2, tm_b2,
                    f_out, te_out):
    f_out[...] = (jnp.dot(species[...], ea_W[...],
                          preferred_element_type=jnp.float32) + ea_b[...])
    proj = (2.0 * jnp.pi) * (t[...] * rff_W[...])
    feats = jnp.concatenate([jnp.cos(proj), jnp.sin(proj)], axis=-1)
    u = _silu(jnp.dot(feats, tm_W1[...], preferred_element_type=jnp.float32)
              + tm_b1[...])
    te_out[...] = (jnp.dot(u, tm_W2[...], preferred_element_type=jnp.float32)
                   + tm_b2[...])


def _nodes_pre(species, t, ea_W, ea_b, rff_W, tm_W1, tm_b1, tm_W2, tm_b2):
    nb = N // BN
    full = lambda shape: pl.BlockSpec(shape, lambda n: (0,) * len(shape))
    return pl.pallas_call(
        _nodes_pre_body,
        grid=(nb,),
        in_specs=[
            pl.BlockSpec((BN, 100), lambda n: (n, 0)),
            pl.BlockSpec((BN, 1), lambda n: (n, 0)),
            full((100, D)), full((1, D)), full((1, D // 2)),
            full((D, D)), full((1, D)), full((D, D)), full((1, D)),
        ],
        out_specs=[
            pl.BlockSpec((BN, D), lambda n: (n, 0)),
            pl.BlockSpec((BN, D), lambda n: (n, 0)),
        ],
        out_shape=[
            jax.ShapeDtypeStruct((NPAD, D), jnp.float32),
            jax.ShapeDtypeStruct((NPAD, D), jnp.float32),
        ],
    )(species, t, ea_W, ea_b, rff_W, tm_W1, tm_b1, tm_W2, tm_b2)


def _ea_body(edge_attrT, W1, b1, W2, b2, ea_out):
    u = _silu(lax.dot_general(edge_attrT[...], W1[...],
                              (((0,), (0,)), ((), ())),
                              preferred_element_type=jnp.float32) + b1[...])
    ea_out[...] = (jnp.dot(u, W2[...], preferred_element_type=jnp.float32)
                   + b2[...])


def _ea_mlp(edge_attrT, W1, b1, W2, b2):
    be = 2048
    nb = (E + be - 1) // be
    full = lambda shape: pl.BlockSpec(shape, lambda n: (0,) * len(shape))
    return pl.pallas_call(
        _ea_body,
        grid=(nb,),
        in_specs=[
            pl.BlockSpec((120, be), lambda n: (0, n)),
            full((120, D)), full((1, D)), full((D, D)), full((1, D)),
        ],
        out_specs=pl.BlockSpec((be, D), lambda n: (n, 0)),
        out_shape=jax.ShapeDtypeStruct((E, D), jnp.float32),
    )(edge_attrT, W1, b1, W2, b2)


BE3 = 2048


def _edges_body(fi, fj, ea, ev, ps_W1, ps_b1, ps_W2, ps_b2,
                pv_W1, pv_b1, pv_W2, pv_b2, out):
    fi_v = fi[...]
    fj_v = fj[...]
    ea_v = ea[...]

    bf = jnp.bfloat16
    fi_b = fi_v.astype(bf)
    fj_b = fj_v.astype(bf)
    ea_b = ea_v.astype(bf)

    def mlp3(W1, b1, W2, b2):
        W1_b = W1[...].astype(bf)
        u = (jnp.dot(fi_b, W1_b[0:D], preferred_element_type=jnp.float32)
             + jnp.dot(fj_b, W1_b[D:2 * D], preferred_element_type=jnp.float32)
             + jnp.dot(ea_b, W1_b[2 * D:3 * D], preferred_element_type=jnp.float32)
             + b1[...])
        return jnp.dot(_silu(u).astype(bf), W2[...].astype(bf),
                       preferred_element_type=jnp.float32) + b2[...]

    rows = (pl.program_id(0) * BE3
            + lax.broadcasted_iota(jnp.int32, (BE3, 1), 0))
    valid = rows < E
    msg = mlp3(ps_W1, ps_b1, ps_W2, ps_b2) * fi_v
    vw = mlp3(pv_W1, pv_b1, pv_W2, pv_b2)
    ev_v = ev[...]
    out[0] = jnp.where(valid, msg, 0.0)
    out[1] = jnp.where(valid, vw * ev_v[:, 0:1], 0.0)
    out[2] = jnp.where(valid, vw * ev_v[:, 1:2], 0.0)
    out[3] = jnp.where(valid, vw * ev_v[:, 2:3], 0.0)


def _edges_mlp(fi, fj, ea, edge_vec, ps_W1, ps_b1, ps_W2, ps_b2,
               pv_W1, pv_b1, pv_W2, pv_b2):
    nb = EPAD // BE3
    full = lambda shape: pl.BlockSpec(shape, lambda n: (0,) * len(shape))
    return pl.pallas_call(
        _edges_body,
        grid=(nb,),
        in_specs=[
            pl.BlockSpec((BE3, D), lambda n: (n, 0)),
            pl.BlockSpec((BE3, D), lambda n: (n, 0)),
            pl.BlockSpec((BE3, D), lambda n: (jnp.minimum(n, 78), 0)),
            pl.BlockSpec((BE3, 3), lambda n: (jnp.minimum(n, 78), 0)),
            full((3 * D, D)), full((1, D)), full((D, D)), full((1, D)),
            full((3 * D, D)), full((1, D)), full((D, D)), full((1, D)),
        ],
        out_specs=pl.BlockSpec((4, BE3, D), lambda n: (0, n, 0)),
        out_shape=jax.ShapeDtypeStruct((4, EPAD, D), jnp.float32),
    )(fi, fj, ea, edge_vec, ps_W1, ps_b1, ps_W2, ps_b2,
      pv_W1, pv_b1, pv_W2, pv_b2)


def _h0_body(f, agg, te, W1, b1, W2, b2, h0_out):
    u = (jnp.dot(f[...], W1[0:D], preferred_element_type=jnp.float32)
         + jnp.dot(agg[...], W1[D:2 * D], preferred_element_type=jnp.float32)
         + b1[...])
    h0_out[...] = (jnp.dot(_silu(u), W2[...],
                           preferred_element_type=jnp.float32)
                   + b2[...] + te[...])


def _h0_mlp(f, agg, te, W1, b1, W2, b2):
    nb = N // BN
    full = lambda shape: pl.BlockSpec(shape, lambda n: (0,) * len(shape))
    return pl.pallas_call(
        _h0_body,
        grid=(nb,),
        in_specs=[
            pl.BlockSpec((BN, D), lambda n: (n, 0)),
            pl.BlockSpec((BN, D), lambda n: (n, 0)),
            pl.BlockSpec((BN, D), lambda n: (n, 0)),
            full((2 * D, D)), full((1, D)), full((D, D)), full((1, D)),
        ],
        out_specs=pl.BlockSpec((BN, D), lambda n: (n, 0)),
        out_shape=jax.ShapeDtypeStruct((N, D), jnp.float32),
    )(f, agg, te, W1, b1, W2, b2)


def _sc_gather(f, ig, jg):
    mesh = plsc.VectorSubcoreMesh(core_axis_name="c", subcore_axis_name="s")

    @functools.partial(
        pl.kernel,
        mesh=mesh,
        out_type=[jax.ShapeDtypeStruct((EPAD, D), jnp.float32),
                  jax.ShapeDtypeStruct((EPAD, D), jnp.float32)],
        scratch_types=[
            pltpu.VMEM_SHARED((NPAD, D), jnp.float32),
            pltpu.VMEM((GPW, G), jnp.int32),
            pltpu.VMEM((GPW, G), jnp.int32),
            pltpu.VMEM((G, D), jnp.float32),
            pltpu.VMEM((G, D), jnp.float32),
            pltpu.SemaphoreType.DMA,
            pltpu.SemaphoreType.DMA,
        ],
    )
    def k(f_hbm, ig_hbm, jg_hbm, fi_hbm, fj_hbm,
          fsh, iv, jv, ri, rj, gs, ws):
        c = lax.axis_index("c")
        s = lax.axis_index("s")
        wid = s * 2 + c
        start = wid * GPW
        pltpu.sync_copy(f_hbm.at[pl.ds(s * NROWS, NROWS)],
                        fsh.at[pl.ds(s * NROWS, NROWS)])
        pltpu.sync_copy(ig_hbm.at[pl.ds(start, GPW)], iv)
        pltpu.sync_copy(jg_hbm.at[pl.ds(start, GPW)], jv)
        plsc.subcore_barrier()

        def drain_wb():
            pltpu.make_async_copy(ri, fi_hbm.at[pl.ds(0, G)], ws).wait()
            pltpu.make_async_copy(rj, fj_hbm.at[pl.ds(0, G)], ws).wait()

        def body(t, carry):
            @pl.when(t > 0)
            def _():
                drain_wb()

            ci = pltpu.async_copy(fsh.at[iv.at[t]], ri, gs)
            cj = pltpu.async_copy(fsh.at[jv.at[t]], rj, gs)
            ci.wait()
            pltpu.async_copy(ri, fi_hbm.at[pl.ds((start + t) * G, G)], ws)
            cj.wait()
            pltpu.async_copy(rj, fj_hbm.at[pl.ds((start + t) * G, G)], ws)
            return carry

        lax.fori_loop(0, GPW, body, 0)
        drain_wb()

    return k(f, ig, jg)


NSUB = 16
GPSU = GPAD // NSUB
NPAD = 10240
NROWS = NPAD // NSUB
ZR = 128


def _sc_scatter(stacked, jg):
    mesh = plsc.VectorSubcoreMesh(core_axis_name="c", subcore_axis_name="s")

    @functools.partial(
        pl.kernel,
        mesh=mesh,
        out_type=jax.ShapeDtypeStruct((4, NPAD, D), jnp.float32),
        scratch_types=[
            pltpu.VMEM((GPSU, G), jnp.int32),
            pltpu.VMEM((G, D), jnp.float32),
            pltpu.VMEM((G, D), jnp.float32),
            pltpu.VMEM_SHARED((NPAD, D), jnp.float32),
            pltpu.SemaphoreType.DMA,
            pltpu.SemaphoreType.DMA,
        ],
    )
    def k(st_hbm, jg_hbm, out_hbm, idxv, dbufA, dbufB, acc, lsA, lsB):
        c = lax.axis_index("c")
        s = lax.axis_index("s")

        start = s * GPSU
        pltpu.sync_copy(jg_hbm.at[pl.ds(start, GPSU)], idxv)
        NIT = GPSU // 2

        for p in range(2):
            pass_idx = c * 2 + p

            def zb(tt, carry):
                dbufA[tt // 8, pl.ds((tt % 8) * 16, 16)] = jnp.zeros(
                    (16,), jnp.float32)
                return carry

            lax.fori_loop(0, ZR * 8, zb, 0)
            for q in range(NROWS // ZR):
                pltpu.sync_copy(dbufA,
                                acc.at[pl.ds(s * NROWS + q * ZR, ZR)])
            plsc.subcore_barrier()

            pltpu.async_copy(st_hbm.at[pass_idx, pl.ds(start * G, G)],
                             dbufA, lsA)

            def body(t, carry):
                rp = 2 * t
                rq = 2 * t + 1
                pltpu.make_async_copy(st_hbm.at[0, pl.ds(0, G)],
                                      dbufA, lsA).wait()
                pltpu.async_copy(
                    st_hbm.at[pass_idx, pl.ds((start + rq) * G, G)],
                    dbufB, lsB)
                pltpu.sync_copy(dbufA, acc.at[idxv.at[rp]], add=True)
                pltpu.make_async_copy(st_hbm.at[0, pl.ds(0, G)],
                                      dbufB, lsB).wait()

                @pl.when(t < NIT - 1)
                def _():
                    pltpu.async_copy(
                        st_hbm.at[pass_idx, pl.ds((start + rp + 2) * G, G)],
                        dbufA, lsA)

                pltpu.sync_copy(dbufB, acc.at[idxv.at[rq]], add=True)
                return carry

            lax.fori_loop(0, NIT, body, 0)
            plsc.subcore_barrier()
            pltpu.sync_copy(acc.at[pl.ds(s * NROWS, NROWS)],
                            out_hbm.at[pass_idx, pl.ds(s * NROWS, NROWS)])
            plsc.subcore_barrier()

    return k(stacked, jg)


def kernel(species, edge_index, edge_attr, edge_vec, t,
           ea_W, ea_b, eb_W1, eb_b1, eb_W2, eb_b2,
           ps_W1, ps_b1, ps_W2, ps_b2,
           ph_W1, ph_b1, ph_W2, ph_b2,
           pv_W1, pv_b1, pv_W2, pv_b2,
           rff_W, tm_W1, tm_b1, tm_W2, tm_b2):
    r = lambda b: b.reshape(1, -1)
    i = edge_index[0]
    j = edge_index[1]

    f, te = _nodes_pre(species, t, ea_W, r(ea_b), rff_W,
                       tm_W1, r(tm_b1), tm_W2, r(tm_b2))
    ea = _ea_mlp(edge_attr.T, eb_W1, r(eb_b1), eb_W2, r(eb_b2))

    pad = EPAD - E
    ig = jnp.pad(i, (0, pad)).reshape(GPAD, G)
    jg = jnp.pad(j, (0, pad)).reshape(GPAD, G)
    fi, fj = _sc_gather(f, ig, jg)

    stacked = _edges_mlp(fi, fj, ea, edge_vec,
                         ps_W1, r(ps_b1), ps_W2, r(ps_b2),
                         pv_W1, r(pv_b1), pv_W2, r(pv_b2))

    segs = _sc_scatter(stacked, jg)
    agg = segs[0]
    v0 = jnp.transpose(segs[1:4, :N], (1, 2, 0))

    h0 = _h0_mlp(f, agg, te, ph_W1, r(ph_b1), ph_W2, r(ph_b2))
    return (h0, v0, ea)


# speedup vs baseline: 25.1763x; 1.0082x over previous
import functools

import jax
import jax.numpy as jnp
from jax import lax
from jax.experimental import pallas as pl
from jax.experimental.pallas import tpu as pltpu
from jax.experimental.pallas import tpu_sc as plsc

N = 10000
E = 160000
D = 128

BN = 2000
BE = 2000

G = 128
NW = 32
NGROUPS = E // G
GPAD = ((NGROUPS + NW - 1) // NW) * NW
EPAD = GPAD * G
GPW = GPAD // NW


def _silu(x):
    return x * jax.nn.sigmoid(x)


def _nodes_pre_body(species, t, ea_W, ea_b, rff_W, tm_W1, tm_b1, tm_W2, tm_b2,
                    f_out, te_out):
    f_out[...] = (jnp.dot(species[...], ea_W[...],
                          preferred_element_type=jnp.float32) + ea_b[...])
    proj = (2.0 * jnp.pi) * (t[...] * rff_W[...])
    feats = jnp.concatenate([jnp.cos(proj), jnp.sin(proj)], axis=-1)
    u = _silu(jnp.dot(feats, tm_W1[...], preferred_element_type=jnp.float32)
              + tm_b1[...])
    te_out[...] = (jnp.dot(u, tm_W2[...], preferred_element_type=jnp.float32)
                   + tm_b2[...])


def _nodes_pre(species, t, ea_W, ea_b, rff_W, tm_W1, tm_b1, tm_W2, tm_b2):
    nb = N // BN
    full = lambda shape: pl.BlockSpec(shape, lambda n: (0,) * len(shape))
    return pl.pallas_call(
        _nodes_pre_body,
        grid=(nb,),
        in_specs=[
            pl.BlockSpec((BN, 100), lambda n: (n, 0)),
            pl.BlockSpec((BN, 1), lambda n: (n, 0)),
            full((100, D)), full((1, D)), full((1, D // 2)),
            full((D, D)), full((1, D)), full((D, D)), full((1, D)),
        ],
        out_specs=[
            pl.BlockSpec((BN, D), lambda n: (n, 0)),
            pl.BlockSpec((BN, D), lambda n: (n, 0)),
        ],
        out_shape=[
            jax.ShapeDtypeStruct((NPAD, D), jnp.float32),
            jax.ShapeDtypeStruct((NPAD, D), jnp.float32),
        ],
    )(species, t, ea_W, ea_b, rff_W, tm_W1, tm_b1, tm_W2, tm_b2)


def _ea_body(edge_attrT, W1, b1, W2, b2, ea_out):
    u = _silu(lax.dot_general(edge_attrT[...], W1[...],
                              (((0,), (0,)), ((), ())),
                              preferred_element_type=jnp.float32) + b1[...])
    ea_out[...] = (jnp.dot(u, W2[...], preferred_element_type=jnp.float32)
                   + b2[...])


def _ea_mlp(edge_attrT, W1, b1, W2, b2):
    be = 2048
    nb = (E + be - 1) // be
    full = lambda shape: pl.BlockSpec(shape, lambda n: (0,) * len(shape))
    return pl.pallas_call(
        _ea_body,
        grid=(nb,),
        in_specs=[
            pl.BlockSpec((120, be), lambda n: (0, n)),
            full((120, D)), full((1, D)), full((D, D)), full((1, D)),
        ],
        out_specs=pl.BlockSpec((be, D), lambda n: (n, 0)),
        out_shape=jax.ShapeDtypeStruct((E, D), jnp.float32),
    )(edge_attrT, W1, b1, W2, b2)


BE3 = 2048


def _edges_body(fi, fj, ea, ev, ps_W1, ps_b1, ps_W2, ps_b2,
                pv_W1, pv_b1, pv_W2, pv_b2, out):
    fi_v = fi[...]
    fj_v = fj[...]
    ea_v = ea[...]

    def mlp3(W1, b1, W2, b2):
        u = (jnp.dot(fi_v, W1[0:D], preferred_element_type=jnp.float32)
             + jnp.dot(fj_v, W1[D:2 * D], preferred_element_type=jnp.float32)
             + jnp.dot(ea_v, W1[2 * D:3 * D], preferred_element_type=jnp.float32)
             + b1[...])
        return jnp.dot(_silu(u), W2[...],
                       preferred_element_type=jnp.float32) + b2[...]

    rows = (pl.program_id(0) * BE3
            + lax.broadcasted_iota(jnp.int32, (BE3, 1), 0))
    valid = rows < E
    msg = mlp3(ps_W1, ps_b1, ps_W2, ps_b2) * fi_v
    vw = mlp3(pv_W1, pv_b1, pv_W2, pv_b2)
    ev_v = ev[...]
    out[0] = jnp.where(valid, msg, 0.0)
    out[1] = jnp.where(valid, vw * ev_v[:, 0:1], 0.0)
    out[2] = jnp.where(valid, vw * ev_v[:, 1:2], 0.0)
    out[3] = jnp.where(valid, vw * ev_v[:, 2:3], 0.0)


def _edges_mlp(fi, fj, ea, edge_vec, ps_W1, ps_b1, ps_W2, ps_b2,
               pv_W1, pv_b1, pv_W2, pv_b2):
    nb = EPAD // BE3
    full = lambda shape: pl.BlockSpec(shape, lambda n: (0,) * len(shape))
    return pl.pallas_call(
        _edges_body,
        grid=(nb,),
        in_specs=[
            pl.BlockSpec((BE3, D), lambda n: (n, 0)),
            pl.BlockSpec((BE3, D), lambda n: (n, 0)),
            pl.BlockSpec((BE3, D), lambda n: (jnp.minimum(n, 78), 0)),
            pl.BlockSpec((BE3, 3), lambda n: (jnp.minimum(n, 78), 0)),
            full((3 * D, D)), full((1, D)), full((D, D)), full((1, D)),
            full((3 * D, D)), full((1, D)), full((D, D)), full((1, D)),
        ],
        out_specs=pl.BlockSpec((4, BE3, D), lambda n: (0, n, 0)),
        out_shape=jax.ShapeDtypeStruct((4, EPAD, D), jnp.float32),
    )(fi, fj, ea, edge_vec, ps_W1, ps_b1, ps_W2, ps_b2,
      pv_W1, pv_b1, pv_W2, pv_b2)


def _h0_body(f, agg, te, W1, b1, W2, b2, h0_out):
    u = (jnp.dot(f[...], W1[0:D], preferred_element_type=jnp.float32)
         + jnp.dot(agg[...], W1[D:2 * D], preferred_element_type=jnp.float32)
         + b1[...])
    h0_out[...] = (jnp.dot(_silu(u), W2[...],
                           preferred_element_type=jnp.float32)
                   + b2[...] + te[...])


def _h0_mlp(f, agg, te, W1, b1, W2, b2):
    nb = N // BN
    full = lambda shape: pl.BlockSpec(shape, lambda n: (0,) * len(shape))
    return pl.pallas_call(
        _h0_body,
        grid=(nb,),
        in_specs=[
            pl.BlockSpec((BN, D), lambda n: (n, 0)),
            pl.BlockSpec((BN, D), lambda n: (n, 0)),
            pl.BlockSpec((BN, D), lambda n: (n, 0)),
            full((2 * D, D)), full((1, D)), full((D, D)), full((1, D)),
        ],
        out_specs=pl.BlockSpec((BN, D), lambda n: (n, 0)),
        out_shape=jax.ShapeDtypeStruct((N, D), jnp.float32),
    )(f, agg, te, W1, b1, W2, b2)


def _sc_gather(f, ig, jg):
    mesh = plsc.VectorSubcoreMesh(core_axis_name="c", subcore_axis_name="s")

    @functools.partial(
        pl.kernel,
        mesh=mesh,
        out_type=[jax.ShapeDtypeStruct((EPAD, D), jnp.float32),
                  jax.ShapeDtypeStruct((EPAD, D), jnp.float32)],
        scratch_types=[
            pltpu.VMEM_SHARED((NPAD, D), jnp.float32),
            pltpu.VMEM((GPW, G), jnp.int32),
            pltpu.VMEM((GPW, G), jnp.int32),
            pltpu.VMEM((G, D), jnp.float32),
            pltpu.VMEM((G, D), jnp.float32),
            pltpu.SemaphoreType.DMA,
            pltpu.SemaphoreType.DMA,
        ],
    )
    def k(f_hbm, ig_hbm, jg_hbm, fi_hbm, fj_hbm,
          fsh, iv, jv, ri, rj, gs, ws):
        c = lax.axis_index("c")
        s = lax.axis_index("s")
        wid = s * 2 + c
        start = wid * GPW
        pltpu.sync_copy(f_hbm.at[pl.ds(s * NROWS, NROWS)],
                        fsh.at[pl.ds(s * NROWS, NROWS)])
        pltpu.sync_copy(ig_hbm.at[pl.ds(start, GPW)], iv)
        pltpu.sync_copy(jg_hbm.at[pl.ds(start, GPW)], jv)
        plsc.subcore_barrier()

        def drain_wb():
            pltpu.make_async_copy(ri, fi_hbm.at[pl.ds(0, G)], ws).wait()
            pltpu.make_async_copy(rj, fj_hbm.at[pl.ds(0, G)], ws).wait()

        def body(t, carry):
            @pl.when(t > 0)
            def _():
                drain_wb()

            ci = pltpu.async_copy(fsh.at[iv.at[t]], ri, gs)
            cj = pltpu.async_copy(fsh.at[jv.at[t]], rj, gs)
            ci.wait()
            pltpu.async_copy(ri, fi_hbm.at[pl.ds((start + t) * G, G)], ws)
            cj.wait()
            pltpu.async_copy(rj, fj_hbm.at[pl.ds((start + t) * G, G)], ws)
            return carry

        lax.fori_loop(0, GPW, body, 0)
        drain_wb()

    return k(f, ig, jg)


NSUB = 16
GPSU = GPAD // NSUB
NPAD = 10240
NROWS = NPAD // NSUB
ZR = 128


def _sc_scatter(stacked, jg):
    mesh = plsc.VectorSubcoreMesh(core_axis_name="c", subcore_axis_name="s")

    @functools.partial(
        pl.kernel,
        mesh=mesh,
        out_type=jax.ShapeDtypeStruct((4, NPAD, D), jnp.float32),
        scratch_types=[
            pltpu.VMEM((GPSU, G), jnp.int32),
            pltpu.VMEM((G, D), jnp.float32),
            pltpu.VMEM((G, D), jnp.float32),
            pltpu.VMEM_SHARED((NPAD, D), jnp.float32),
            pltpu.SemaphoreType.DMA,
            pltpu.SemaphoreType.DMA,
        ],
    )
    def k(st_hbm, jg_hbm, out_hbm, idxv, dbufA, dbufB, acc, lsA, lsB):
        c = lax.axis_index("c")
        s = lax.axis_index("s")

        start = s * GPSU
        pltpu.sync_copy(jg_hbm.at[pl.ds(start, GPSU)], idxv)
        NIT = GPSU // 2

        for p in range(2):
            pass_idx = c * 2 + p

            def zb(tt, carry):
                dbufA[tt // 8, pl.ds((tt % 8) * 16, 16)] = jnp.zeros(
                    (16,), jnp.float32)
                return carry

            lax.fori_loop(0, ZR * 8, zb, 0)
            for q in range(NROWS // ZR):
                pltpu.sync_copy(dbufA,
                                acc.at[pl.ds(s * NROWS + q * ZR, ZR)])
            plsc.subcore_barrier()

            pltpu.async_copy(st_hbm.at[pass_idx, pl.ds(start * G, G)],
                             dbufA, lsA)

            def body(t, carry):
                rp = 2 * t
                rq = 2 * t + 1
                pltpu.make_async_copy(st_hbm.at[0, pl.ds(0, G)],
                                      dbufA, lsA).wait()
                pltpu.async_copy(
                    st_hbm.at[pass_idx, pl.ds((start + rq) * G, G)],
                    dbufB, lsB)
                pltpu.sync_copy(dbufA, acc.at[idxv.at[rp]], add=True)
                pltpu.make_async_copy(st_hbm.at[0, pl.ds(0, G)],
                                      dbufB, lsB).wait()

                @pl.when(t < NIT - 1)
                def _():
                    pltpu.async_copy(
                        st_hbm.at[pass_idx, pl.ds((start + rp + 2) * G, G)],
                        dbufA, lsA)

                pltpu.sync_copy(dbufB, acc.at[idxv.at[rq]], add=True)
                return carry

            lax.fori_loop(0, NIT, body, 0)
            plsc.subcore_barrier()
            pltpu.sync_copy(acc.at[pl.ds(s * NROWS, NROWS)],
                            out_hbm.at[pass_idx, pl.ds(s * NROWS, NROWS)])
            plsc.subcore_barrier()

    return k(stacked, jg)


def kernel(species, edge_index, edge_attr, edge_vec, t,
           ea_W, ea_b, eb_W1, eb_b1, eb_W2, eb_b2,
           ps_W1, ps_b1, ps_W2, ps_b2,
           ph_W1, ph_b1, ph_W2, ph_b2,
           pv_W1, pv_b1, pv_W2, pv_b2,
           rff_W, tm_W1, tm_b1, tm_W2, tm_b2):
    r = lambda b: b.reshape(1, -1)
    i = edge_index[0]
    j = edge_index[1]

    f, te = _nodes_pre(species, t, ea_W, r(ea_b), rff_W,
                       tm_W1, r(tm_b1), tm_W2, r(tm_b2))
    ea = _ea_mlp(edge_attr.T, eb_W1, r(eb_b1), eb_W2, r(eb_b2))

    pad = EPAD - E
    ig = jnp.pad(i, (0, pad)).reshape(GPAD, G)
    jg = jnp.pad(j, (0, pad)).reshape(GPAD, G)
    fi, fj = _sc_gather(f, ig, jg)

    stacked = _edges_mlp(fi, fj, ea, edge_vec,
                         ps_W1, r(ps_b1), ps_W2, r(ps_b2),
                         pv_W1, r(pv_b1), pv_W2, r(pv_b2))

    segs = _sc_scatter(stacked, jg)
    agg = segs[0]
    v0 = jnp.transpose(segs[1:4, :N], (1, 2, 0))

    h0 = _h0_mlp(f, agg, te, ph_W1, r(ph_b1), ph_W2, r(ph_b2))
    return (h0, v0, ea)


# speedup vs baseline: 26.0455x; 1.0345x over previous
import functools

import jax
import jax.numpy as jnp
from jax import lax
from jax.experimental import pallas as pl
from jax.experimental.pallas import tpu as pltpu
from jax.experimental.pallas import tpu_sc as plsc

N = 10000
E = 160000
D = 128

BN = 2000
BE = 2000

G = 128
NW = 32
NGROUPS = E // G
GPAD = ((NGROUPS + NW - 1) // NW) * NW
EPAD = GPAD * G
GPW = GPAD // NW


def _silu(x):
    return x * jax.nn.sigmoid(x)


def _nodes_pre_body(species, t, ea_W, ea_b, rff_W, tm_W1, tm_b1, tm_W2, tm_b2,
                    f_out, te_out):
    f_out[...] = (jnp.dot(species[...], ea_W[...],
                          preferred_element_type=jnp.float32) + ea_b[...])
    proj = (2.0 * jnp.pi) * (t[...] * rff_W[...])
    feats = jnp.concatenate([jnp.cos(proj), jnp.sin(proj)], axis=-1)
    u = _silu(jnp.dot(feats, tm_W1[...], preferred_element_type=jnp.float32)
              + tm_b1[...])
    te_out[...] = (jnp.dot(u, tm_W2[...], preferred_element_type=jnp.float32)
                   + tm_b2[...])


def _nodes_pre(species, t, ea_W, ea_b, rff_W, tm_W1, tm_b1, tm_W2, tm_b2):
    nb = N // BN
    full = lambda shape: pl.BlockSpec(shape, lambda n: (0,) * len(shape))
    return pl.pallas_call(
        _nodes_pre_body,
        grid=(nb,),
        in_specs=[
            pl.BlockSpec((BN, 100), lambda n: (n, 0)),
            pl.BlockSpec((BN, 1), lambda n: (n, 0)),
            full((100, D)), full((1, D)), full((1, D // 2)),
            full((D, D)), full((1, D)), full((D, D)), full((1, D)),
        ],
        out_specs=[
            pl.BlockSpec((BN, D), lambda n: (n, 0)),
            pl.BlockSpec((BN, D), lambda n: (n, 0)),
        ],
        out_shape=[
            jax.ShapeDtypeStruct((NPAD, D), jnp.float32),
            jax.ShapeDtypeStruct((NPAD, D), jnp.float32),
        ],
    )(species, t, ea_W, ea_b, rff_W, tm_W1, tm_b1, tm_W2, tm_b2)


def _ea_body(edge_attrT, W1, b1, W2, b2, ea_out):
    u = _silu(lax.dot_general(edge_attrT[...], W1[...],
                              (((0,), (0,)), ((), ())),
                              preferred_element_type=jnp.float32) + b1[...])
    ea_out[...] = (jnp.dot(u, W2[...], preferred_element_type=jnp.float32)
                   + b2[...])


def _ea_mlp(edge_attrT, W1, b1, W2, b2):
    be = 2048
    nb = (E + be - 1) // be
    full = lambda shape: pl.BlockSpec(shape, lambda n: (0,) * len(shape))
    return pl.pallas_call(
        _ea_body,
        grid=(nb,),
        in_specs=[
            pl.BlockSpec((120, be), lambda n: (0, n)),
            full((120, D)), full((1, D)), full((D, D)), full((1, D)),
        ],
        out_specs=pl.BlockSpec((be, D), lambda n: (n, 0)),
        out_shape=jax.ShapeDtypeStruct((E, D), jnp.float32),
    )(edge_attrT, W1, b1, W2, b2)


BE3 = 2048


def _edges_body(fi, fj, ea, ev, ps_W1, ps_b1, ps_W2, ps_b2,
                pv_W1, pv_b1, pv_W2, pv_b2, out, *, blk_lo):
    fi_v = fi[...]
    fj_v = fj[...]
    ea_v = ea[...]

    def mlp3(W1, b1, W2, b2):
        u = (jnp.dot(fi_v, W1[0:D], preferred_element_type=jnp.float32)
             + jnp.dot(fj_v, W1[D:2 * D], preferred_element_type=jnp.float32)
             + jnp.dot(ea_v, W1[2 * D:3 * D], preferred_element_type=jnp.float32)
             + b1[...])
        return jnp.dot(_silu(u), W2[...],
                       preferred_element_type=jnp.float32) + b2[...]

    rows = ((pl.program_id(0) + blk_lo) * BE3
            + lax.broadcasted_iota(jnp.int32, (BE3, 1), 0))
    valid = rows < E
    msg = mlp3(ps_W1, ps_b1, ps_W2, ps_b2) * fi_v
    vw = mlp3(pv_W1, pv_b1, pv_W2, pv_b2)
    ev_v = ev[...]
    out[0] = jnp.where(valid, msg, 0.0)
    out[1] = jnp.where(valid, vw * ev_v[:, 0:1], 0.0)
    out[2] = jnp.where(valid, vw * ev_v[:, 1:2], 0.0)
    out[3] = jnp.where(valid, vw * ev_v[:, 2:3], 0.0)


NBH = (EPAD // BE3) // 2


def _edges_mlp(fi, fj, ea, edge_vec, ps_W1, ps_b1, ps_W2, ps_b2,
               pv_W1, pv_b1, pv_W2, pv_b2, blk_lo):
    full = lambda shape: pl.BlockSpec(shape, lambda n: (0,) * len(shape))
    body = functools.partial(_edges_body, blk_lo=blk_lo)
    return pl.pallas_call(
        body,
        grid=(NBH,),
        in_specs=[
            pl.BlockSpec((BE3, D), lambda n: (n + blk_lo, 0)),
            pl.BlockSpec((BE3, D), lambda n: (n + blk_lo, 0)),
            pl.BlockSpec((BE3, D), lambda n: (jnp.minimum(n + blk_lo, 78), 0)),
            pl.BlockSpec((BE3, 3), lambda n: (jnp.minimum(n + blk_lo, 78), 0)),
            full((3 * D, D)), full((1, D)), full((D, D)), full((1, D)),
            full((3 * D, D)), full((1, D)), full((D, D)), full((1, D)),
        ],
        out_specs=pl.BlockSpec((4, BE3, D), lambda n: (0, n, 0)),
        out_shape=jax.ShapeDtypeStruct((4, NBH * BE3, D), jnp.float32),
    )(fi, fj, ea, edge_vec, ps_W1, ps_b1, ps_W2, ps_b2,
      pv_W1, pv_b1, pv_W2, pv_b2)


def _h0_body(f, agg, te, W1, b1, W2, b2, h0_out):
    u = (jnp.dot(f[...], W1[0:D], preferred_element_type=jnp.float32)
         + jnp.dot(agg[...], W1[D:2 * D], preferred_element_type=jnp.float32)
         + b1[...])
    h0_out[...] = (jnp.dot(_silu(u), W2[...],
                           preferred_element_type=jnp.float32)
                   + b2[...] + te[...])


def _h0_mlp(f, agg, te, W1, b1, W2, b2):
    nb = N // BN
    full = lambda shape: pl.BlockSpec(shape, lambda n: (0,) * len(shape))
    return pl.pallas_call(
        _h0_body,
        grid=(nb,),
        in_specs=[
            pl.BlockSpec((BN, D), lambda n: (n, 0)),
            pl.BlockSpec((BN, D), lambda n: (n, 0)),
            pl.BlockSpec((BN, D), lambda n: (n, 0)),
            full((2 * D, D)), full((1, D)), full((D, D)), full((1, D)),
        ],
        out_specs=pl.BlockSpec((BN, D), lambda n: (n, 0)),
        out_shape=jax.ShapeDtypeStruct((N, D), jnp.float32),
    )(f, agg, te, W1, b1, W2, b2)


def _sc_gather(f, ig, jg):
    mesh = plsc.VectorSubcoreMesh(core_axis_name="c", subcore_axis_name="s")

    @functools.partial(
        pl.kernel,
        mesh=mesh,
        out_type=[jax.ShapeDtypeStruct((EPAD, D), jnp.float32),
                  jax.ShapeDtypeStruct((EPAD, D), jnp.float32)],
        scratch_types=[
            pltpu.VMEM_SHARED((NPAD, D), jnp.float32),
            pltpu.VMEM((GPW, G), jnp.int32),
            pltpu.VMEM((GPW, G), jnp.int32),
            pltpu.VMEM((G, D), jnp.float32),
            pltpu.VMEM((G, D), jnp.float32),
            pltpu.SemaphoreType.DMA,
            pltpu.SemaphoreType.DMA,
        ],
    )
    def k(f_hbm, ig_hbm, jg_hbm, fi_hbm, fj_hbm,
          fsh, iv, jv, ri, rj, gs, ws):
        c = lax.axis_index("c")
        s = lax.axis_index("s")
        wid = s * 2 + c
        start = wid * GPW
        pltpu.sync_copy(f_hbm.at[pl.ds(s * NROWS, NROWS)],
                        fsh.at[pl.ds(s * NROWS, NROWS)])
        pltpu.sync_copy(ig_hbm.at[pl.ds(start, GPW)], iv)
        pltpu.sync_copy(jg_hbm.at[pl.ds(start, GPW)], jv)
        plsc.subcore_barrier()

        def drain_wb():
            pltpu.make_async_copy(ri, fi_hbm.at[pl.ds(0, G)], ws).wait()
            pltpu.make_async_copy(rj, fj_hbm.at[pl.ds(0, G)], ws).wait()

        def body(t, carry):
            @pl.when(t > 0)
            def _():
                drain_wb()

            ci = pltpu.async_copy(fsh.at[iv.at[t]], ri, gs)
            cj = pltpu.async_copy(fsh.at[jv.at[t]], rj, gs)
            ci.wait()
            pltpu.async_copy(ri, fi_hbm.at[pl.ds((start + t) * G, G)], ws)
            cj.wait()
            pltpu.async_copy(rj, fj_hbm.at[pl.ds((start + t) * G, G)], ws)
            return carry

        lax.fori_loop(0, GPW, body, 0)
        drain_wb()

    return k(f, ig, jg)


NSUB = 16
GHALF = GPAD // 2
GPSU = GHALF // NSUB
NPAD = 10240
NROWS = NPAD // NSUB
ZR = 128


def _sc_scatter(stacked, jg, base_group, init):
    mesh = plsc.VectorSubcoreMesh(core_axis_name="c", subcore_axis_name="s")
    have_init = init is not None
    scratch = [
        pltpu.VMEM((GPSU, G), jnp.int32),
        pltpu.VMEM((G, D), jnp.float32),
        pltpu.VMEM((G, D), jnp.float32),
        pltpu.VMEM_SHARED((NPAD, D), jnp.float32),
        pltpu.SemaphoreType.DMA,
        pltpu.SemaphoreType.DMA,
    ]

    @functools.partial(
        pl.kernel,
        mesh=mesh,
        out_type=jax.ShapeDtypeStruct((4, NPAD, D), jnp.float32),
        scratch_types=scratch,
    )
    def k(st_hbm, jg_hbm, *rest):
        if have_init:
            init_hbm, out_hbm, idxv, dbufA, dbufB, acc, lsA, lsB = rest
        else:
            out_hbm, idxv, dbufA, dbufB, acc, lsA, lsB = rest
        c = lax.axis_index("c")
        s = lax.axis_index("s")

        start = s * GPSU
        pltpu.sync_copy(jg_hbm.at[pl.ds(base_group + start, GPSU)], idxv)
        NIT = GPSU // 2

        for p in range(2):
            pass_idx = c * 2 + p

            if have_init:
                pltpu.sync_copy(init_hbm.at[pass_idx,
                                            pl.ds(s * NROWS, NROWS)],
                                acc.at[pl.ds(s * NROWS, NROWS)])
            else:
                def zb(tt, carry):
                    dbufA[tt // 8, pl.ds((tt % 8) * 16, 16)] = jnp.zeros(
                        (16,), jnp.float32)
                    return carry

                lax.fori_loop(0, ZR * 8, zb, 0)
                for q in range(NROWS // ZR):
                    pltpu.sync_copy(dbufA,
                                    acc.at[pl.ds(s * NROWS + q * ZR, ZR)])
            plsc.subcore_barrier()

            pltpu.async_copy(st_hbm.at[pass_idx, pl.ds(start * G, G)],
                             dbufA, lsA)

            def body(t, carry):
                rp = 2 * t
                rq = 2 * t + 1
                pltpu.make_async_copy(st_hbm.at[0, pl.ds(0, G)],
                                      dbufA, lsA).wait()
                pltpu.async_copy(
                    st_hbm.at[pass_idx, pl.ds((start + rq) * G, G)],
                    dbufB, lsB)
                pltpu.sync_copy(dbufA, acc.at[idxv.at[rp]], add=True)
                pltpu.make_async_copy(st_hbm.at[0, pl.ds(0, G)],
                                      dbufB, lsB).wait()

                @pl.when(t < NIT - 1)
                def _():
                    pltpu.async_copy(
                        st_hbm.at[pass_idx, pl.ds((start + rp + 2) * G, G)],
                        dbufA, lsA)

                pltpu.sync_copy(dbufB, acc.at[idxv.at[rq]], add=True)
                return carry

            lax.fori_loop(0, NIT, body, 0)
            plsc.subcore_barrier()
            pltpu.sync_copy(acc.at[pl.ds(s * NROWS, NROWS)],
                            out_hbm.at[pass_idx, pl.ds(s * NROWS, NROWS)])
            plsc.subcore_barrier()

    if have_init:
        return k(stacked, jg, init)
    return k(stacked, jg)


def kernel(species, edge_index, edge_attr, edge_vec, t,
           ea_W, ea_b, eb_W1, eb_b1, eb_W2, eb_b2,
           ps_W1, ps_b1, ps_W2, ps_b2,
           ph_W1, ph_b1, ph_W2, ph_b2,
           pv_W1, pv_b1, pv_W2, pv_b2,
           rff_W, tm_W1, tm_b1, tm_W2, tm_b2):
    r = lambda b: b.reshape(1, -1)
    i = edge_index[0]
    j = edge_index[1]

    f, te = _nodes_pre(species, t, ea_W, r(ea_b), rff_W,
                       tm_W1, r(tm_b1), tm_W2, r(tm_b2))
    ea = _ea_mlp(edge_attr.T, eb_W1, r(eb_b1), eb_W2, r(eb_b2))

    pad = EPAD - E
    ig = jnp.pad(i, (0, pad)).reshape(GPAD, G)
    jg = jnp.pad(j, (0, pad)).reshape(GPAD, G)
    fi, fj = _sc_gather(f, ig, jg)

    st1 = _edges_mlp(fi, fj, ea, edge_vec,
                     ps_W1, r(ps_b1), ps_W2, r(ps_b2),
                     pv_W1, r(pv_b1), pv_W2, r(pv_b2), 0)
    seg1 = _sc_scatter(st1, jg, 0, None)
    st2 = _edges_mlp(fi, fj, ea, edge_vec,
                     ps_W1, r(ps_b1), ps_W2, r(ps_b2),
                     pv_W1, r(pv_b1), pv_W2, r(pv_b2), NBH)
    segs = _sc_scatter(st2, jg, GHALF, seg1)
    agg = segs[0]
    v0 = jnp.transpose(segs[1:4, :N], (1, 2, 0))

    h0 = _h0_mlp(f, agg, te, ph_W1, r(ph_b1), ph_W2, r(ph_b2))
    return (h0, v0, ea)


# speedup vs baseline: 27.6357x; 1.0611x over previous
import functools

import jax
import jax.numpy as jnp
from jax import lax
from jax.experimental import pallas as pl
from jax.experimental.pallas import tpu as pltpu
from jax.experimental.pallas import tpu_sc as plsc

N = 10000
E = 160000
D = 128

BN = 2000
BE = 2000

G = 128
NW = 32
NGROUPS = E // G
GPAD = ((NGROUPS + NW - 1) // NW) * NW
EPAD = GPAD * G
GPW = GPAD // NW


def _silu(x):
    return x * jax.nn.sigmoid(x)


def _nodes_pre_body(species, t, ea_W, ea_b, rff_W, tm_W1, tm_b1, tm_W2, tm_b2,
                    f_out, te_out):
    f_out[...] = (jnp.dot(species[...], ea_W[...],
                          preferred_element_type=jnp.float32) + ea_b[...])
    proj = (2.0 * jnp.pi) * (t[...] * rff_W[...])
    feats = jnp.concatenate([jnp.cos(proj), jnp.sin(proj)], axis=-1)
    u = _silu(jnp.dot(feats, tm_W1[...], preferred_element_type=jnp.float32)
              + tm_b1[...])
    te_out[...] = (jnp.dot(u, tm_W2[...], preferred_element_type=jnp.float32)
                   + tm_b2[...])


def _nodes_pre(species, t, ea_W, ea_b, rff_W, tm_W1, tm_b1, tm_W2, tm_b2):
    nb = N // BN
    full = lambda shape: pl.BlockSpec(shape, lambda n: (0,) * len(shape))
    return pl.pallas_call(
        _nodes_pre_body,
        grid=(nb,),
        in_specs=[
            pl.BlockSpec((BN, 100), lambda n: (n, 0)),
            pl.BlockSpec((BN, 1), lambda n: (n, 0)),
            full((100, D)), full((1, D)), full((1, D // 2)),
            full((D, D)), full((1, D)), full((D, D)), full((1, D)),
        ],
        out_specs=[
            pl.BlockSpec((BN, D), lambda n: (n, 0)),
            pl.BlockSpec((BN, D), lambda n: (n, 0)),
        ],
        out_shape=[
            jax.ShapeDtypeStruct((NPAD, D), jnp.float32),
            jax.ShapeDtypeStruct((NPAD, D), jnp.float32),
        ],
    )(species, t, ea_W, ea_b, rff_W, tm_W1, tm_b1, tm_W2, tm_b2)


def _ea_body(edge_attrT, W1, b1, W2, b2, ea_out):
    u = _silu(lax.dot_general(edge_attrT[...], W1[...],
                              (((0,), (0,)), ((), ())),
                              preferred_element_type=jnp.float32) + b1[...])
    ea_out[...] = (jnp.dot(u, W2[...], preferred_element_type=jnp.float32)
                   + b2[...])


def _ea_mlp(edge_attrT, W1, b1, W2, b2):
    be = 2048
    nb = (E + be - 1) // be
    full = lambda shape: pl.BlockSpec(shape, lambda n: (0,) * len(shape))
    return pl.pallas_call(
        _ea_body,
        grid=(nb,),
        in_specs=[
            pl.BlockSpec((120, be), lambda n: (0, n)),
            full((120, D)), full((1, D)), full((D, D)), full((1, D)),
        ],
        out_specs=pl.BlockSpec((be, D), lambda n: (n, 0)),
        out_shape=jax.ShapeDtypeStruct((E, D), jnp.float32),
    )(edge_attrT, W1, b1, W2, b2)


BE3 = 2048


def _edges_body(fi, fj, ea, evT, ps_W1, ps_b1, ps_W2, ps_b2,
                pv_W1, pv_b1, pv_W2, pv_b2, out, *, blk_lo):
    fi_v = fi[...]
    fj_v = fj[...]
    ea_v = ea[...]

    def mlp3(W1, b1, W2, b2):
        u = (jnp.dot(fi_v, W1[0:D], preferred_element_type=jnp.float32)
             + jnp.dot(fj_v, W1[D:2 * D], preferred_element_type=jnp.float32)
             + jnp.dot(ea_v, W1[2 * D:3 * D], preferred_element_type=jnp.float32)
             + b1[...])
        return jnp.dot(_silu(u), W2[...],
                       preferred_element_type=jnp.float32) + b2[...]

    rows = ((pl.program_id(0) + blk_lo) * BE3
            + lax.broadcasted_iota(jnp.int32, (BE3, 1), 0))
    valid = rows < E
    msg = mlp3(ps_W1, ps_b1, ps_W2, ps_b2) * fi_v
    vw = mlp3(pv_W1, pv_b1, pv_W2, pv_b2)
    eye3 = (lax.broadcasted_iota(jnp.int32, (3, 3), 0)
            == lax.broadcasted_iota(jnp.int32, (3, 3), 1)).astype(jnp.float32)
    ev_v = lax.dot_general(evT[...], eye3, (((0,), (0,)), ((), ())),
                           preferred_element_type=jnp.float32)
    out[0] = jnp.where(valid, msg, 0.0)
    out[1] = jnp.where(valid, vw * ev_v[:, 0:1], 0.0)
    out[2] = jnp.where(valid, vw * ev_v[:, 1:2], 0.0)
    out[3] = jnp.where(valid, vw * ev_v[:, 2:3], 0.0)


NBH = (EPAD // BE3) // 2


def _edges_mlp(fi, fj, ea, edge_vec, ps_W1, ps_b1, ps_W2, ps_b2,
               pv_W1, pv_b1, pv_W2, pv_b2, blk_lo):
    full = lambda shape: pl.BlockSpec(shape, lambda n: (0,) * len(shape))
    body = functools.partial(_edges_body, blk_lo=blk_lo)
    return pl.pallas_call(
        body,
        grid=(NBH,),
        in_specs=[
            pl.BlockSpec((BE3, D), lambda n: (n + blk_lo, 0)),
            pl.BlockSpec((BE3, D), lambda n: (n + blk_lo, 0)),
            pl.BlockSpec((BE3, D), lambda n: (jnp.minimum(n + blk_lo, 78), 0)),
            pl.BlockSpec((3, BE3), lambda n: (0, jnp.minimum(n + blk_lo, 78))),
            full((3 * D, D)), full((1, D)), full((D, D)), full((1, D)),
            full((3 * D, D)), full((1, D)), full((D, D)), full((1, D)),
        ],
        out_specs=pl.BlockSpec((4, BE3, D), lambda n: (0, n, 0)),
        out_shape=jax.ShapeDtypeStruct((4, NBH * BE3, D), jnp.float32),
    )(fi, fj, ea, edge_vec, ps_W1, ps_b1, ps_W2, ps_b2,
      pv_W1, pv_b1, pv_W2, pv_b2)


def _h0_body(f, agg, te, W1, b1, W2, b2, h0_out):
    u = (jnp.dot(f[...], W1[0:D], preferred_element_type=jnp.float32)
         + jnp.dot(agg[...], W1[D:2 * D], preferred_element_type=jnp.float32)
         + b1[...])
    h0_out[...] = (jnp.dot(_silu(u), W2[...],
                           preferred_element_type=jnp.float32)
                   + b2[...] + te[...])


def _h0_mlp(f, agg, te, W1, b1, W2, b2):
    nb = N // BN
    full = lambda shape: pl.BlockSpec(shape, lambda n: (0,) * len(shape))
    return pl.pallas_call(
        _h0_body,
        grid=(nb,),
        in_specs=[
            pl.BlockSpec((BN, D), lambda n: (n, 0)),
            pl.BlockSpec((BN, D), lambda n: (n, 0)),
            pl.BlockSpec((BN, D), lambda n: (n, 0)),
            full((2 * D, D)), full((1, D)), full((D, D)), full((1, D)),
        ],
        out_specs=pl.BlockSpec((BN, D), lambda n: (n, 0)),
        out_shape=jax.ShapeDtypeStruct((N, D), jnp.float32),
    )(f, agg, te, W1, b1, W2, b2)


def _sc_gather(f, ig, jg):
    mesh = plsc.VectorSubcoreMesh(core_axis_name="c", subcore_axis_name="s")

    @functools.partial(
        pl.kernel,
        mesh=mesh,
        out_type=[jax.ShapeDtypeStruct((EPAD, D), jnp.float32),
                  jax.ShapeDtypeStruct((EPAD, D), jnp.float32)],
        scratch_types=[
            pltpu.VMEM_SHARED((NPAD, D), jnp.float32),
            pltpu.VMEM((GPW, G), jnp.int32),
            pltpu.VMEM((GPW, G), jnp.int32),
            pltpu.VMEM((G, D), jnp.float32),
            pltpu.VMEM((G, D), jnp.float32),
            pltpu.SemaphoreType.DMA,
            pltpu.SemaphoreType.DMA,
        ],
    )
    def k(f_hbm, ig_hbm, jg_hbm, fi_hbm, fj_hbm,
          fsh, iv, jv, ri, rj, gs, ws):
        c = lax.axis_index("c")
        s = lax.axis_index("s")
        wid = s * 2 + c
        start = wid * GPW
        pltpu.sync_copy(f_hbm.at[pl.ds(s * NROWS, NROWS)],
                        fsh.at[pl.ds(s * NROWS, NROWS)])
        pltpu.sync_copy(ig_hbm.at[pl.ds(start, GPW)], iv)
        pltpu.sync_copy(jg_hbm.at[pl.ds(start, GPW)], jv)
        plsc.subcore_barrier()

        def drain_wb():
            pltpu.make_async_copy(ri, fi_hbm.at[pl.ds(0, G)], ws).wait()
            pltpu.make_async_copy(rj, fj_hbm.at[pl.ds(0, G)], ws).wait()

        def body(t, carry):
            @pl.when(t > 0)
            def _():
                drain_wb()

            ci = pltpu.async_copy(fsh.at[iv.at[t]], ri, gs)
            cj = pltpu.async_copy(fsh.at[jv.at[t]], rj, gs)
            ci.wait()
            pltpu.async_copy(ri, fi_hbm.at[pl.ds((start + t) * G, G)], ws)
            cj.wait()
            pltpu.async_copy(rj, fj_hbm.at[pl.ds((start + t) * G, G)], ws)
            return carry

        lax.fori_loop(0, GPW, body, 0)
        drain_wb()

    return k(f, ig, jg)


NSUB = 16
GHALF = GPAD // 2
GPSU = GHALF // NSUB
NPAD = 10240
NROWS = NPAD // NSUB
ZR = 128


def _sc_scatter(stacked, jg, base_group, init):
    mesh = plsc.VectorSubcoreMesh(core_axis_name="c", subcore_axis_name="s")
    have_init = init is not None
    scratch = [
        pltpu.VMEM((GPSU, G), jnp.int32),
        pltpu.VMEM((G, D), jnp.float32),
        pltpu.VMEM((G, D), jnp.float32),
        pltpu.VMEM_SHARED((NPAD, D), jnp.float32),
        pltpu.SemaphoreType.DMA,
        pltpu.SemaphoreType.DMA,
    ]

    @functools.partial(
        pl.kernel,
        mesh=mesh,
        out_type=jax.ShapeDtypeStruct((4, NPAD, D), jnp.float32),
        scratch_types=scratch,
    )
    def k(st_hbm, jg_hbm, *rest):
        if have_init:
            init_hbm, out_hbm, idxv, dbufA, dbufB, acc, lsA, lsB = rest
        else:
            out_hbm, idxv, dbufA, dbufB, acc, lsA, lsB = rest
        c = lax.axis_index("c")
        s = lax.axis_index("s")

        start = s * GPSU
        pltpu.sync_copy(jg_hbm.at[pl.ds(base_group + start, GPSU)], idxv)
        NIT = GPSU // 2

        for p in range(2):
            pass_idx = c * 2 + p

            if have_init:
                pltpu.sync_copy(init_hbm.at[pass_idx,
                                            pl.ds(s * NROWS, NROWS)],
                                acc.at[pl.ds(s * NROWS, NROWS)])
            else:
                def zb(tt, carry):
                    dbufA[tt // 8, pl.ds((tt % 8) * 16, 16)] = jnp.zeros(
                        (16,), jnp.float32)
                    return carry

                lax.fori_loop(0, ZR * 8, zb, 0)
                for q in range(NROWS // ZR):
                    pltpu.sync_copy(dbufA,
                                    acc.at[pl.ds(s * NROWS + q * ZR, ZR)])
            plsc.subcore_barrier()

            pltpu.async_copy(st_hbm.at[pass_idx, pl.ds(start * G, G)],
                             dbufA, lsA)

            def body(t, carry):
                rp = 2 * t
                rq = 2 * t + 1
                pltpu.make_async_copy(st_hbm.at[0, pl.ds(0, G)],
                                      dbufA, lsA).wait()
                pltpu.async_copy(
                    st_hbm.at[pass_idx, pl.ds((start + rq) * G, G)],
                    dbufB, lsB)
                pltpu.sync_copy(dbufA, acc.at[idxv.at[rp]], add=True)
                pltpu.make_async_copy(st_hbm.at[0, pl.ds(0, G)],
                                      dbufB, lsB).wait()

                @pl.when(t < NIT - 1)
                def _():
                    pltpu.async_copy(
                        st_hbm.at[pass_idx, pl.ds((start + rp + 2) * G, G)],
                        dbufA, lsA)

                pltpu.sync_copy(dbufB, acc.at[idxv.at[rq]], add=True)
                return carry

            lax.fori_loop(0, NIT, body, 0)
            plsc.subcore_barrier()
            pltpu.sync_copy(acc.at[pl.ds(s * NROWS, NROWS)],
                            out_hbm.at[pass_idx, pl.ds(s * NROWS, NROWS)])
            plsc.subcore_barrier()

    if have_init:
        return k(stacked, jg, init)
    return k(stacked, jg)


def kernel(species, edge_index, edge_attr, edge_vec, t,
           ea_W, ea_b, eb_W1, eb_b1, eb_W2, eb_b2,
           ps_W1, ps_b1, ps_W2, ps_b2,
           ph_W1, ph_b1, ph_W2, ph_b2,
           pv_W1, pv_b1, pv_W2, pv_b2,
           rff_W, tm_W1, tm_b1, tm_W2, tm_b2):
    r = lambda b: b.reshape(1, -1)
    i = edge_index[0]
    j = edge_index[1]

    f, te = _nodes_pre(species, t, ea_W, r(ea_b), rff_W,
                       tm_W1, r(tm_b1), tm_W2, r(tm_b2))
    ea = _ea_mlp(edge_attr.T, eb_W1, r(eb_b1), eb_W2, r(eb_b2))

    pad = EPAD - E
    ig = jnp.pad(i, (0, pad)).reshape(GPAD, G)
    jg = jnp.pad(j, (0, pad)).reshape(GPAD, G)
    fi, fj = _sc_gather(f, ig, jg)

    edge_vecT = edge_vec.T
    st1 = _edges_mlp(fi, fj, ea, edge_vecT,
                     ps_W1, r(ps_b1), ps_W2, r(ps_b2),
                     pv_W1, r(pv_b1), pv_W2, r(pv_b2), 0)
    seg1 = _sc_scatter(st1, jg, 0, None)
    st2 = _edges_mlp(fi, fj, ea, edge_vecT,
                     ps_W1, r(ps_b1), ps_W2, r(ps_b2),
                     pv_W1, r(pv_b1), pv_W2, r(pv_b2), NBH)
    segs = _sc_scatter(st2, jg, GHALF, seg1)
    agg = segs[0]
    v0 = jnp.transpose(segs[1:4, :N], (1, 2, 0))

    h0 = _h0_mlp(f, agg, te, ph_W1, r(ph_b1), ph_W2, r(ph_b2))
    return (h0, v0, ea)
